# Initial kernel scaffold; baseline (speedup 1.0000x reference)
import jax
import jax.numpy as jnp
from jax.experimental import pallas as pl


def kernel(pos, pos_u, edge_index_tt, u_src, u_dst, hist_x, history_u, us, W_alpha, b_alpha, W_beta, b_beta, W_hist, b_hist, W_c2h, b_c2h, W_h2x, b_h2x):
    raise NotImplementedError("write your pallas kernel here")


# speedup vs baseline: 5.7370x; 5.7370x over previous
import functools

import jax
import jax.numpy as jnp
from jax import lax
from jax.experimental import pallas as pl
from jax.experimental.pallas import tpu as pltpu
from jax.experimental.pallas import tpu_sc as plsc

N_G = 8000
N_C = 2000
N = N_G + N_C
NP = 10240
N_U = 2000
E_TT = 320000
E_U = 40000
POS = 3
HIST = 20
UH = 8
T = 4
DH = 128

NC = 2
NS = 16
NW = NC * NS
CH = 128

ETT_W = 10240
ETT_P = NW * ETT_W
ETT_CH = ETT_W // CH
EU_W = 1280
EU_P = NW * EU_W
EU_CH = EU_W // CH

ROWS_W = NP // NS
GRP = 2048
GRP_CH = GRP // CH
N_GRP = ETT_W // GRP


def _prep_body(histx_ref, whist_ref, bhist_ref, ufull_ref, wc2h_ref, bc2h_ref,
               pos_ref, wa_ref, posu_ref, wb_ref,
               h0_ref, cu_ref, gsa_ref, gda_ref, gsb_ref, gdb_ref):
    h0_ref[...] = (jnp.dot(histx_ref[...], whist_ref[...],
                           preferred_element_type=jnp.float32)
                   + bhist_ref[0, :][None, :])
    for t in range(T):
        u_win = ufull_ref[:, t:t + UH]
        cu_ref[t] = (jnp.dot(u_win, wc2h_ref[...],
                             preferred_element_type=jnp.float32)
                     + bc2h_ref[0, :][None, :])
    pos = pos_ref[...]
    gsa_ref[...] = (pos[:, 0:1] * wa_ref[0, 0] + pos[:, 1:2] * wa_ref[0, 1]
                    + pos[:, 2:3] * wa_ref[0, 2])
    gda_ref[...] = (pos[:, 0:1] * wa_ref[0, 3] + pos[:, 1:2] * wa_ref[0, 4]
                    + pos[:, 2:3] * wa_ref[0, 5] + wa_ref[0, 6])
    posu = posu_ref[...]
    gsb_ref[...] = (posu[:, 0:1] * wb_ref[0, 0] + posu[:, 1:2] * wb_ref[0, 1]
                    + posu[:, 2:3] * wb_ref[0, 2])
    gdb_ref[...] = (pos[:, 0:1] * wb_ref[0, 3] + pos[:, 1:2] * wb_ref[0, 4]
                    + pos[:, 2:3] * wb_ref[0, 5] + wb_ref[0, 6])


def _prep(histx_p, w_hist, b_hist, u_full, w_c2h, b_c2h, pos_p, wa, pos_u, wb):
    return pl.pallas_call(
        _prep_body,
        out_shape=(
            jax.ShapeDtypeStruct((NP, DH), jnp.float32),
            jax.ShapeDtypeStruct((T, N_U, DH), jnp.float32),
            jax.ShapeDtypeStruct((NP, 1), jnp.float32),
            jax.ShapeDtypeStruct((NP, 1), jnp.float32),
            jax.ShapeDtypeStruct((N_U, 1), jnp.float32),
            jax.ShapeDtypeStruct((NP, 1), jnp.float32),
        ),
    )(histx_p, w_hist, b_hist, u_full, w_c2h, b_c2h, pos_p, wa, pos_u, wb)


def _tanh16(x):
    e = jnp.exp(-2.0 * jnp.abs(x))
    t = (1.0 - e) / (1.0 + e)
    return jnp.where(x < 0.0, -t, t)


def _gate_body(gsa_hbm, gda_hbm, gsb_hbm, gdb_hbm,
               src_hbm, dst_hbm, usrc_hbm, udst_hbm,
               alpha_hbm, beta_hbm,
               gsa_v, gda_v, gsb_v, gdb_v,
               src_v, dst_v, usrc_v, udst_v, alpha_v, beta_v):
    c = lax.axis_index("c")
    s = lax.axis_index("s")
    wid = s * NC + c
    iota = lax.iota(jnp.int32, 16)

    pltpu.sync_copy(gsa_hbm, gsa_v)
    pltpu.sync_copy(gda_hbm, gda_v)
    pltpu.sync_copy(gsb_hbm, gsb_v)
    pltpu.sync_copy(gdb_hbm, gdb_v)

    base = wid * ETT_W
    pltpu.sync_copy(src_hbm.at[pl.ds(base, ETT_W)], src_v)
    pltpu.sync_copy(dst_hbm.at[pl.ds(base, ETT_W)], dst_v)
    ubase = wid * EU_W
    pltpu.sync_copy(usrc_hbm.at[pl.ds(ubase, EU_W)], usrc_v)
    pltpu.sync_copy(udst_hbm.at[pl.ds(ubase, EU_W)], udst_v)

    def tt_step(i, _):
        si = src_v[pl.ds(i * 16, 16)]
        di = dst_v[pl.ds(i * 16, 16)]
        x = (plsc.load_gather(gsa_v, [si]) + plsc.load_gather(gda_v, [di]))
        a = 0.05 * _tanh16(x)
        gidx = base + i * 16 + iota
        alpha_v[pl.ds(i * 16, 16)] = jnp.where(gidx < E_TT, a, 0.0)
        return _

    lax.fori_loop(0, ETT_W // 16, tt_step, None)
    pltpu.sync_copy(alpha_v, alpha_hbm.at[pl.ds(base, ETT_W)])

    def u_step(i, _):
        si = usrc_v[pl.ds(i * 16, 16)]
        di = udst_v[pl.ds(i * 16, 16)]
        x = (plsc.load_gather(gsb_v, [si]) + plsc.load_gather(gdb_v, [di]))
        b = 0.05 * _tanh16(x)
        gidx = ubase + i * 16 + iota
        beta_v[pl.ds(i * 16, 16)] = jnp.where(gidx < E_U, b, 0.0)
        return _

    lax.fori_loop(0, EU_W // 16, u_step, None)
    pltpu.sync_copy(beta_v, beta_hbm.at[pl.ds(ubase, EU_W)])


def _gates(gsa, gda, gsb, gdb, srcp, dstp, usrcp, udstp):
    mesh = plsc.VectorSubcoreMesh(core_axis_name="c", subcore_axis_name="s")
    f = functools.partial(
        pl.kernel,
        out_type=(
            jax.ShapeDtypeStruct((ETT_P,), jnp.float32),
            jax.ShapeDtypeStruct((EU_P,), jnp.float32),
        ),
        mesh=mesh,
        compiler_params=pltpu.CompilerParams(needs_layout_passes=False),
        scratch_types=[
            pltpu.VMEM((NP,), jnp.float32),
            pltpu.VMEM((NP,), jnp.float32),
            pltpu.VMEM((N_U,), jnp.float32),
            pltpu.VMEM((NP,), jnp.float32),
            pltpu.VMEM((ETT_W,), jnp.int32),
            pltpu.VMEM((ETT_W,), jnp.int32),
            pltpu.VMEM((EU_W,), jnp.int32),
            pltpu.VMEM((EU_W,), jnp.int32),
            pltpu.VMEM((ETT_W,), jnp.float32),
            pltpu.VMEM((EU_W,), jnp.float32),
        ],
    )(_gate_body)
    return f(gsa, gda, gsb, gdb, srcp, dstp, usrcp, udstp)


def _step_body(h_hbm, cu_hbm, alpha_hbm, beta_hbm,
               src_hbm, dst2_hbm, usrc_hbm, udst2_hbm,
               partial_hbm,
               acc, srcg_v, dstg_v, alphag_v, usrc_v, udst2_v, beta_v,
               rows_v, sem):
    c = lax.axis_index("c")
    s = lax.axis_index("s")
    wid = s * NC + c

    zero16 = jnp.zeros((16,), jnp.float32)

    def zrow(i, _):
        for g in range(DH // 16):
            rows_v[i, pl.ds(g * 16, 16)] = zero16
        return _

    lax.fori_loop(0, CH, zrow, None)
    for j in range(ROWS_W // CH):
        pltpu.sync_copy(rows_v, acc.at[pl.ds(s * ROWS_W + j * CH, CH)])

    ubase = wid * EU_W
    pltpu.sync_copy(usrc_hbm.at[pl.ds(ubase, EU_W)], usrc_v)
    pltpu.sync_copy(beta_hbm.at[pl.ds(ubase, EU_W)], beta_v)
    pltpu.sync_copy(udst2_hbm.at[wid], udst2_v)

    plsc.subcore_barrier()

    def scale_scatter(table_hbm, idx_slice, d2_row, w_v, wbase):
        pltpu.async_copy(table_hbm.at[idx_slice], rows_v, sem).wait()

        def edge4(m, _):
            for q in range(4):
                e = m * 4 + q
                a = plsc.load_gather(
                    w_v, [jnp.full((16,), wbase + e, jnp.int32)])
                for g in range(DH // 16):
                    rows_v[e, pl.ds(g * 16, 16)] = (
                        rows_v[e, pl.ds(g * 16, 16)] * a)
            return _

        lax.fori_loop(0, CH // 4, edge4, None)
        pltpu.async_copy(rows_v, acc.at[d2_row], sem, add=True).wait()

    def group(g, _):
        base = wid * ETT_W + g * GRP
        pltpu.sync_copy(src_hbm.at[pl.ds(base, GRP)], srcg_v)
        pltpu.sync_copy(alpha_hbm.at[pl.ds(base, GRP)], alphag_v)
        pltpu.sync_copy(dst2_hbm.at[wid, pl.ds(g * GRP_CH, GRP_CH)], dstg_v)

        def chunk(k, _):
            scale_scatter(h_hbm, srcg_v.at[pl.ds(k * CH, CH)],
                          dstg_v.at[k], alphag_v, k * CH)
            return _

        lax.fori_loop(0, GRP_CH, chunk, None)
        return _

    lax.fori_loop(0, N_GRP, group, None)

    def uchunk(k, _):
        scale_scatter(cu_hbm, usrc_v.at[pl.ds(k * CH, CH)],
                      udst2_v.at[k], beta_v, k * CH)
        return _

    lax.fori_loop(0, EU_CH, uchunk, None)

    plsc.subcore_barrier()
    for j in range(ROWS_W // CH):
        r = s * ROWS_W + j * CH
        pltpu.sync_copy(acc.at[pl.ds(r, CH)], partial_hbm.at[c, pl.ds(r, CH)])


def _step(h, cu_t, alpha, beta, srcp, dst2, usrcp, udst2):
    mesh = plsc.VectorSubcoreMesh(core_axis_name="c", subcore_axis_name="s")
    f = functools.partial(
        pl.kernel,
        out_type=jax.ShapeDtypeStruct((NC, NP, DH), jnp.float32),
        mesh=mesh,
        compiler_params=pltpu.CompilerParams(needs_layout_passes=False),
        scratch_types=[
            pltpu.VMEM_SHARED((NP, DH), jnp.float32),
            pltpu.VMEM((GRP,), jnp.int32),
            pltpu.VMEM((GRP_CH, CH), jnp.int32),
            pltpu.VMEM((GRP,), jnp.float32),
            pltpu.VMEM((EU_W,), jnp.int32),
            pltpu.VMEM((EU_CH, CH), jnp.int32),
            pltpu.VMEM((EU_W,), jnp.float32),
            pltpu.VMEM((CH, DH), jnp.float32),
            pltpu.SemaphoreType.DMA,
        ],
    )(_step_body)
    return f(h, cu_t, alpha, beta, srcp, dst2, usrcp, udst2)


def _combine_body(p_ref, w_ref, b_ref, h_ref, xs_ref):
    h = p_ref[0] + p_ref[1]
    h_ref[...] = h
    xs_ref[...] = (jnp.sum(h * w_ref[0, :][None, :], axis=1, keepdims=True)
                   + b_ref[0, 0])


def _combine(partial, w_h2x_row, b_h2x):
    return pl.pallas_call(
        _combine_body,
        out_shape=(
            jax.ShapeDtypeStruct((NP, DH), jnp.float32),
            jax.ShapeDtypeStruct((NP, 1), jnp.float32),
        ),
    )(partial, w_h2x_row, b_h2x)


def kernel(pos, pos_u, edge_index_tt, u_src, u_dst, hist_x, history_u, us,
           W_alpha, b_alpha, W_beta, b_beta, W_hist, b_hist,
           W_c2h, b_c2h, W_h2x, b_h2x):
    f32 = jnp.float32
    pos_p = jnp.pad(pos, ((0, NP - N), (0, 0)))
    histx_p = jnp.pad(hist_x, ((0, NP - N), (0, 0)))
    u_full = jnp.concatenate([history_u, us], axis=1)

    src = edge_index_tt[0]
    dst = edge_index_tt[1]
    pad_tt = ETT_P - E_TT
    pad_ids = (jnp.arange(pad_tt, dtype=jnp.int32) % N)
    srcp = jnp.concatenate([src, pad_ids])
    dstp = jnp.concatenate([dst, pad_ids])
    pad_u = EU_P - E_U
    upad_s = (jnp.arange(pad_u, dtype=jnp.int32) % N_U)
    upad_d = (jnp.arange(pad_u, dtype=jnp.int32) % N)
    usrcp = jnp.concatenate([u_src, upad_s])
    udstp = jnp.concatenate([u_dst, upad_d])
    dst2 = dstp.reshape(NW, ETT_CH, CH)
    udst2 = udstp.reshape(NW, EU_CH, CH)

    wa = jnp.concatenate(
        [W_alpha[:, 0], b_alpha, jnp.zeros((1,), f32)]).reshape(1, 8)
    wb = jnp.concatenate(
        [W_beta[:, 0], b_beta, jnp.zeros((1,), f32)]).reshape(1, 8)

    h0, cu, gsa, gda, gsb, gdb = _prep(
        histx_p, W_hist, b_hist.reshape(1, DH), u_full, W_c2h,
        b_c2h.reshape(1, DH), pos_p, wa, pos_u, wb)

    alpha, beta = _gates(
        gsa.reshape(NP), gda.reshape(NP), gsb.reshape(N_U), gdb.reshape(NP),
        srcp, dstp, usrcp, udstp)

    w_row = W_h2x[:, 0].reshape(1, DH)
    b11 = b_h2x.reshape(1, 1)

    h = h0
    xs_cols = []
    for t in range(T):
        partial = _step(h, cu[t], alpha, beta, srcp, dst2, usrcp, udst2)
        h, xs_t = _combine(partial, w_row, b11)
        xs_cols.append(xs_t)

    xs = jnp.concatenate(xs_cols, axis=1)[:N]
    return xs


# speedup vs baseline: 8.4370x; 1.4706x over previous
import functools

import jax
import jax.numpy as jnp
from jax import lax
from jax.experimental import pallas as pl
from jax.experimental.pallas import tpu as pltpu
from jax.experimental.pallas import tpu_sc as plsc

N_G = 8000
N_C = 2000
N = N_G + N_C
NP = 10240
N_U = 2000
E_TT = 320000
E_U = 40000
POS = 3
HIST = 20
UH = 8
T = 4
DH = 128

NC = 2
NS = 16
NW = NC * NS
CH = 128

ETT_W = 10240
ETT_P = NW * ETT_W
ETT_CH = ETT_W // CH
EU_W = 1280
EU_P = NW * EU_W
EU_CH = EU_W // CH

ROWS_W = NP // NS
GRP = 2048
GRP_CH = GRP // CH
N_GRP = ETT_W // GRP


def _prep_body(histx_ref, whist_ref, bhist_ref, ufull_ref, wc2h_ref, bc2h_ref,
               pos_ref, wa_ref, posu_ref, wb_ref,
               h0_ref, cu_ref, gsa_ref, gda_ref, gsb_ref, gdb_ref):
    h0_ref[...] = (jnp.dot(histx_ref[...], whist_ref[...],
                           preferred_element_type=jnp.float32)
                   + bhist_ref[0, :][None, :])
    for t in range(T):
        u_win = ufull_ref[:, t:t + UH]
        cu_ref[t] = (jnp.dot(u_win, wc2h_ref[...],
                             preferred_element_type=jnp.float32)
                     + bc2h_ref[0, :][None, :])
    pos = pos_ref[...]
    gsa_ref[...] = (pos[:, 0:1] * wa_ref[0, 0] + pos[:, 1:2] * wa_ref[0, 1]
                    + pos[:, 2:3] * wa_ref[0, 2])
    gda_ref[...] = (pos[:, 0:1] * wa_ref[0, 3] + pos[:, 1:2] * wa_ref[0, 4]
                    + pos[:, 2:3] * wa_ref[0, 5] + wa_ref[0, 6])
    posu = posu_ref[...]
    gsb_ref[...] = (posu[:, 0:1] * wb_ref[0, 0] + posu[:, 1:2] * wb_ref[0, 1]
                    + posu[:, 2:3] * wb_ref[0, 2])
    gdb_ref[...] = (pos[:, 0:1] * wb_ref[0, 3] + pos[:, 1:2] * wb_ref[0, 4]
                    + pos[:, 2:3] * wb_ref[0, 5] + wb_ref[0, 6])


def _prep(histx_p, w_hist, b_hist, u_full, w_c2h, b_c2h, pos_p, wa, pos_u, wb):
    return pl.pallas_call(
        _prep_body,
        out_shape=(
            jax.ShapeDtypeStruct((NP, DH), jnp.float32),
            jax.ShapeDtypeStruct((T, N_U, DH), jnp.float32),
            jax.ShapeDtypeStruct((NP, 1), jnp.float32),
            jax.ShapeDtypeStruct((NP, 1), jnp.float32),
            jax.ShapeDtypeStruct((N_U, 1), jnp.float32),
            jax.ShapeDtypeStruct((NP, 1), jnp.float32),
        ),
    )(histx_p, w_hist, b_hist, u_full, w_c2h, b_c2h, pos_p, wa, pos_u, wb)


def _tanh16(x):
    e = jnp.exp(-2.0 * jnp.abs(x))
    t = (1.0 - e) / (1.0 + e)
    return jnp.where(x < 0.0, -t, t)


def _gate_body(gsa_hbm, gda_hbm, gsb_hbm, gdb_hbm,
               src_hbm, dst_hbm, usrc_hbm, udst_hbm,
               alpha_hbm, beta_hbm,
               gsa_v, gda_v, gsb_v, gdb_v,
               src_v, dst_v, usrc_v, udst_v, alpha_v, beta_v):
    c = lax.axis_index("c")
    s = lax.axis_index("s")
    wid = s * NC + c
    iota = lax.iota(jnp.int32, 16)

    pltpu.sync_copy(gsa_hbm, gsa_v)
    pltpu.sync_copy(gda_hbm, gda_v)
    pltpu.sync_copy(gsb_hbm, gsb_v)
    pltpu.sync_copy(gdb_hbm, gdb_v)

    base = wid * ETT_W
    pltpu.sync_copy(src_hbm.at[pl.ds(base, ETT_W)], src_v)
    pltpu.sync_copy(dst_hbm.at[pl.ds(base, ETT_W)], dst_v)
    ubase = wid * EU_W
    pltpu.sync_copy(usrc_hbm.at[pl.ds(ubase, EU_W)], usrc_v)
    pltpu.sync_copy(udst_hbm.at[pl.ds(ubase, EU_W)], udst_v)

    def tt_step(i, _):
        si = src_v[pl.ds(i * 16, 16)]
        di = dst_v[pl.ds(i * 16, 16)]
        x = (plsc.load_gather(gsa_v, [si]) + plsc.load_gather(gda_v, [di]))
        a = 0.05 * _tanh16(x)
        gidx = base + i * 16 + iota
        alpha_v[pl.ds(i * 16, 16)] = jnp.where(gidx < E_TT, a, 0.0)
        return _

    lax.fori_loop(0, ETT_W // 16, tt_step, None)
    pltpu.sync_copy(alpha_v, alpha_hbm.at[pl.ds(base, ETT_W)])

    def u_step(i, _):
        si = usrc_v[pl.ds(i * 16, 16)]
        di = udst_v[pl.ds(i * 16, 16)]
        x = (plsc.load_gather(gsb_v, [si]) + plsc.load_gather(gdb_v, [di]))
        b = 0.05 * _tanh16(x)
        gidx = ubase + i * 16 + iota
        beta_v[pl.ds(i * 16, 16)] = jnp.where(gidx < E_U, b, 0.0)
        return _

    lax.fori_loop(0, EU_W // 16, u_step, None)
    pltpu.sync_copy(beta_v, beta_hbm.at[pl.ds(ubase, EU_W)])


def _gates(gsa, gda, gsb, gdb, srcp, dstp, usrcp, udstp):
    mesh = plsc.VectorSubcoreMesh(core_axis_name="c", subcore_axis_name="s")
    f = functools.partial(
        pl.kernel,
        out_type=(
            jax.ShapeDtypeStruct((ETT_P,), jnp.float32),
            jax.ShapeDtypeStruct((EU_P,), jnp.float32),
        ),
        mesh=mesh,
        compiler_params=pltpu.CompilerParams(needs_layout_passes=False),
        scratch_types=[
            pltpu.VMEM((NP,), jnp.float32),
            pltpu.VMEM((NP,), jnp.float32),
            pltpu.VMEM((N_U,), jnp.float32),
            pltpu.VMEM((NP,), jnp.float32),
            pltpu.VMEM((ETT_W,), jnp.int32),
            pltpu.VMEM((ETT_W,), jnp.int32),
            pltpu.VMEM((EU_W,), jnp.int32),
            pltpu.VMEM((EU_W,), jnp.int32),
            pltpu.VMEM((ETT_W,), jnp.float32),
            pltpu.VMEM((EU_W,), jnp.float32),
        ],
    )(_gate_body)
    return f(gsa, gda, gsb, gdb, srcp, dstp, usrcp, udstp)


def _step_body(h_hbm, cu_hbm, alpha_hbm, beta_hbm,
               src_hbm, dst2_hbm, usrc_hbm, udst2_hbm,
               partial_hbm,
               acc, srcg_v, dstg_v, alphag_v, usrc_v, udst2_v, beta_v,
               rows_a, rows_b, sem_a, sem_b):
    c = lax.axis_index("c")
    s = lax.axis_index("s")
    wid = s * NC + c

    zero16 = jnp.zeros((16,), jnp.float32)

    def zrow(i, _):
        for g in range(DH // 16):
            rows_a[i, pl.ds(g * 16, 16)] = zero16
        return _

    lax.fori_loop(0, CH, zrow, None)
    for j in range(ROWS_W // CH):
        pltpu.sync_copy(rows_a, acc.at[pl.ds(s * ROWS_W + j * CH, CH)])

    ubase = wid * EU_W
    pltpu.sync_copy(usrc_hbm.at[pl.ds(ubase, EU_W)], usrc_v)
    pltpu.sync_copy(beta_hbm.at[pl.ds(ubase, EU_W)], beta_v)
    pltpu.sync_copy(udst2_hbm.at[wid], udst2_v)

    plsc.subcore_barrier()

    def scale(buf, w_v, wbase):
        def edge4(m, _):
            for q in range(4):
                e = m * 4 + q
                a = plsc.load_gather(
                    w_v, [jnp.full((16,), wbase + e, jnp.int32)])
                for g in range(DH // 16):
                    buf[e, pl.ds(g * 16, 16)] = buf[e, pl.ds(g * 16, 16)] * a
            return _

        lax.fori_loop(0, CH // 4, edge4, None)

    def run_chunks(table_hbm, idx_v, d2_v, w_v, nch):
        def g_copy(k, buf, sem):
            return pltpu.make_async_copy(
                table_hbm.at[idx_v.at[pl.ds(k * CH, CH)]], buf, sem)

        def s_copy(k, buf, sem):
            return pltpu.make_async_copy(buf, acc.at[d2_v.at[k]], sem)

        g_copy(0, rows_a, sem_a).start()

        def pair(m, _):
            k0 = 2 * m

            @pl.when(m > 0)
            def _():
                s_copy(k0 - 1, rows_b, sem_b).wait()

            g_copy(k0, rows_a, sem_a).wait()
            g_copy(k0 + 1, rows_b, sem_b).start()
            scale(rows_a, w_v, k0 * CH)
            s_copy(k0, rows_a, sem_a).start(add=True)
            g_copy(k0 + 1, rows_b, sem_b).wait()
            s_copy(k0, rows_a, sem_a).wait()

            @pl.when(k0 + 2 < nch)
            def _():
                g_copy(k0 + 2, rows_a, sem_a).start()

            scale(rows_b, w_v, (k0 + 1) * CH)
            s_copy(k0 + 1, rows_b, sem_b).start(add=True)
            return _

        lax.fori_loop(0, nch // 2, pair, None)
        s_copy(nch - 1, rows_b, sem_b).wait()

    def group(g, _):
        base = wid * ETT_W + g * GRP
        pltpu.sync_copy(src_hbm.at[pl.ds(base, GRP)], srcg_v)
        pltpu.sync_copy(alpha_hbm.at[pl.ds(base, GRP)], alphag_v)
        pltpu.sync_copy(dst2_hbm.at[wid, pl.ds(g * GRP_CH, GRP_CH)], dstg_v)
        run_chunks(h_hbm, srcg_v, dstg_v, alphag_v, GRP_CH)
        return _

    lax.fori_loop(0, N_GRP, group, None)
    run_chunks(cu_hbm, usrc_v, udst2_v, beta_v, EU_CH)

    plsc.subcore_barrier()
    for j in range(ROWS_W // CH):
        r = s * ROWS_W + j * CH
        pltpu.sync_copy(acc.at[pl.ds(r, CH)], partial_hbm.at[c, pl.ds(r, CH)])


def _step(h, cu_t, alpha, beta, srcp, dst2, usrcp, udst2):
    mesh = plsc.VectorSubcoreMesh(core_axis_name="c", subcore_axis_name="s")
    f = functools.partial(
        pl.kernel,
        out_type=jax.ShapeDtypeStruct((NC, NP, DH), jnp.float32),
        mesh=mesh,
        compiler_params=pltpu.CompilerParams(needs_layout_passes=False),
        scratch_types=[
            pltpu.VMEM_SHARED((NP, DH), jnp.float32),
            pltpu.VMEM((GRP,), jnp.int32),
            pltpu.VMEM((GRP_CH, CH), jnp.int32),
            pltpu.VMEM((GRP,), jnp.float32),
            pltpu.VMEM((EU_W,), jnp.int32),
            pltpu.VMEM((EU_CH, CH), jnp.int32),
            pltpu.VMEM((EU_W,), jnp.float32),
            pltpu.VMEM((CH, DH), jnp.float32),
            pltpu.VMEM((CH, DH), jnp.float32),
            pltpu.SemaphoreType.DMA,
            pltpu.SemaphoreType.DMA,
        ],
    )(_step_body)
    return f(h, cu_t, alpha, beta, srcp, dst2, usrcp, udst2)


def _combine_body(p_ref, w_ref, b_ref, h_ref, xs_ref):
    h = p_ref[0] + p_ref[1]
    h_ref[...] = h
    xs_ref[...] = (jnp.sum(h * w_ref[0, :][None, :], axis=1, keepdims=True)
                   + b_ref[0, 0])


def _combine(partial, w_h2x_row, b_h2x):
    return pl.pallas_call(
        _combine_body,
        out_shape=(
            jax.ShapeDtypeStruct((NP, DH), jnp.float32),
            jax.ShapeDtypeStruct((NP, 1), jnp.float32),
        ),
    )(partial, w_h2x_row, b_h2x)


def kernel(pos, pos_u, edge_index_tt, u_src, u_dst, hist_x, history_u, us,
           W_alpha, b_alpha, W_beta, b_beta, W_hist, b_hist,
           W_c2h, b_c2h, W_h2x, b_h2x):
    f32 = jnp.float32
    pos_p = jnp.pad(pos, ((0, NP - N), (0, 0)))
    histx_p = jnp.pad(hist_x, ((0, NP - N), (0, 0)))
    u_full = jnp.concatenate([history_u, us], axis=1)

    src = edge_index_tt[0]
    dst = edge_index_tt[1]
    pad_tt = ETT_P - E_TT
    pad_ids = (jnp.arange(pad_tt, dtype=jnp.int32) % N)
    srcp = jnp.concatenate([src, pad_ids])
    dstp = jnp.concatenate([dst, pad_ids])
    pad_u = EU_P - E_U
    upad_s = (jnp.arange(pad_u, dtype=jnp.int32) % N_U)
    upad_d = (jnp.arange(pad_u, dtype=jnp.int32) % N)
    usrcp = jnp.concatenate([u_src, upad_s])
    udstp = jnp.concatenate([u_dst, upad_d])
    dst2 = dstp.reshape(NW, ETT_CH, CH)
    udst2 = udstp.reshape(NW, EU_CH, CH)

    wa = jnp.concatenate(
        [W_alpha[:, 0], b_alpha, jnp.zeros((1,), f32)]).reshape(1, 8)
    wb = jnp.concatenate(
        [W_beta[:, 0], b_beta, jnp.zeros((1,), f32)]).reshape(1, 8)

    h0, cu, gsa, gda, gsb, gdb = _prep(
        histx_p, W_hist, b_hist.reshape(1, DH), u_full, W_c2h,
        b_c2h.reshape(1, DH), pos_p, wa, pos_u, wb)

    alpha, beta = _gates(
        gsa.reshape(NP), gda.reshape(NP), gsb.reshape(N_U), gdb.reshape(NP),
        srcp, dstp, usrcp, udstp)

    w_row = W_h2x[:, 0].reshape(1, DH)
    b11 = b_h2x.reshape(1, 1)

    h = h0
    xs_cols = []
    for t in range(T):
        partial = _step(h, cu[t], alpha, beta, srcp, dst2, usrcp, udst2)
        h, xs_t = _combine(partial, w_row, b11)
        xs_cols.append(xs_t)

    xs = jnp.concatenate(xs_cols, axis=1)[:N]
    return xs


# speedup vs baseline: 9.2392x; 1.0951x over previous
import functools

import jax
import jax.numpy as jnp
from jax import lax
from jax.experimental import pallas as pl
from jax.experimental.pallas import tpu as pltpu
from jax.experimental.pallas import tpu_sc as plsc

N_G = 8000
N_C = 2000
N = N_G + N_C
NP = 10240
N_U = 2000
E_TT = 320000
E_U = 40000
POS = 3
HIST = 20
UH = 8
T = 4
DH = 128

NC = 2
NS = 16
NW = NC * NS
CH = 64
NBUF = 4

ETT_W = 10240
ETT_P = NW * ETT_W
ETT_CH = ETT_W // CH
EU_W = 1280
EU_P = NW * EU_W
EU_CH = EU_W // CH

ROWS_W = NP // NS
GRP = 1024
GRP_CH = GRP // CH
N_GRP = ETT_W // GRP


def _prep_body(histx_ref, whist_ref, bhist_ref, ufull_ref, wc2h_ref, bc2h_ref,
               pos_ref, wa_ref, posu_ref, wb_ref,
               h0_ref, cu_ref, gsa_ref, gda_ref, gsb_ref, gdb_ref):
    h0_ref[...] = (jnp.dot(histx_ref[...], whist_ref[...],
                           preferred_element_type=jnp.float32)
                   + bhist_ref[0, :][None, :])
    for t in range(T):
        u_win = ufull_ref[:, t:t + UH]
        cu_ref[t] = (jnp.dot(u_win, wc2h_ref[...],
                             preferred_element_type=jnp.float32)
                     + bc2h_ref[0, :][None, :])
    pos = pos_ref[...]
    gsa_ref[...] = (pos[:, 0:1] * wa_ref[0, 0] + pos[:, 1:2] * wa_ref[0, 1]
                    + pos[:, 2:3] * wa_ref[0, 2])
    gda_ref[...] = (pos[:, 0:1] * wa_ref[0, 3] + pos[:, 1:2] * wa_ref[0, 4]
                    + pos[:, 2:3] * wa_ref[0, 5] + wa_ref[0, 6])
    posu = posu_ref[...]
    gsb_ref[...] = (posu[:, 0:1] * wb_ref[0, 0] + posu[:, 1:2] * wb_ref[0, 1]
                    + posu[:, 2:3] * wb_ref[0, 2])
    gdb_ref[...] = (pos[:, 0:1] * wb_ref[0, 3] + pos[:, 1:2] * wb_ref[0, 4]
                    + pos[:, 2:3] * wb_ref[0, 5] + wb_ref[0, 6])


def _prep(histx_p, w_hist, b_hist, u_full, w_c2h, b_c2h, pos_p, wa, pos_u, wb):
    return pl.pallas_call(
        _prep_body,
        out_shape=(
            jax.ShapeDtypeStruct((NP, DH), jnp.float32),
            jax.ShapeDtypeStruct((T, N_U, DH), jnp.float32),
            jax.ShapeDtypeStruct((NP, 1), jnp.float32),
            jax.ShapeDtypeStruct((NP, 1), jnp.float32),
            jax.ShapeDtypeStruct((N_U, 1), jnp.float32),
            jax.ShapeDtypeStruct((NP, 1), jnp.float32),
        ),
    )(histx_p, w_hist, b_hist, u_full, w_c2h, b_c2h, pos_p, wa, pos_u, wb)


def _tanh16(x):
    e = jnp.exp(-2.0 * jnp.abs(x))
    t = (1.0 - e) / (1.0 + e)
    return jnp.where(x < 0.0, -t, t)


def _gate_body(gsa_hbm, gda_hbm, gsb_hbm, gdb_hbm,
               src_hbm, dst_hbm, usrc_hbm, udst_hbm,
               alpha_hbm, beta_hbm,
               gsa_v, gda_v, gsb_v, gdb_v,
               src_v, dst_v, usrc_v, udst_v, alpha_v, beta_v):
    c = lax.axis_index("c")
    s = lax.axis_index("s")
    wid = s * NC + c
    iota = lax.iota(jnp.int32, 16)

    pltpu.sync_copy(gsa_hbm, gsa_v)
    pltpu.sync_copy(gda_hbm, gda_v)
    pltpu.sync_copy(gsb_hbm, gsb_v)
    pltpu.sync_copy(gdb_hbm, gdb_v)

    base = wid * ETT_W
    pltpu.sync_copy(src_hbm.at[pl.ds(base, ETT_W)], src_v)
    pltpu.sync_copy(dst_hbm.at[pl.ds(base, ETT_W)], dst_v)
    ubase = wid * EU_W
    pltpu.sync_copy(usrc_hbm.at[pl.ds(ubase, EU_W)], usrc_v)
    pltpu.sync_copy(udst_hbm.at[pl.ds(ubase, EU_W)], udst_v)

    def tt_step(i, _):
        si = src_v[pl.ds(i * 16, 16)]
        di = dst_v[pl.ds(i * 16, 16)]
        x = (plsc.load_gather(gsa_v, [si]) + plsc.load_gather(gda_v, [di]))
        a = 0.05 * _tanh16(x)
        gidx = base + i * 16 + iota
        alpha_v[pl.ds(i * 16, 16)] = jnp.where(gidx < E_TT, a, 0.0)
        return _

    lax.fori_loop(0, ETT_W // 16, tt_step, None)
    pltpu.sync_copy(alpha_v, alpha_hbm.at[pl.ds(base, ETT_W)])

    def u_step(i, _):
        si = usrc_v[pl.ds(i * 16, 16)]
        di = udst_v[pl.ds(i * 16, 16)]
        x = (plsc.load_gather(gsb_v, [si]) + plsc.load_gather(gdb_v, [di]))
        b = 0.05 * _tanh16(x)
        gidx = ubase + i * 16 + iota
        beta_v[pl.ds(i * 16, 16)] = jnp.where(gidx < E_U, b, 0.0)
        return _

    lax.fori_loop(0, EU_W // 16, u_step, None)
    pltpu.sync_copy(beta_v, beta_hbm.at[pl.ds(ubase, EU_W)])


def _gates(gsa, gda, gsb, gdb, srcp, dstp, usrcp, udstp):
    mesh = plsc.VectorSubcoreMesh(core_axis_name="c", subcore_axis_name="s")
    f = functools.partial(
        pl.kernel,
        out_type=(
            jax.ShapeDtypeStruct((ETT_P,), jnp.float32),
            jax.ShapeDtypeStruct((EU_P,), jnp.float32),
        ),
        mesh=mesh,
        compiler_params=pltpu.CompilerParams(needs_layout_passes=False),
        scratch_types=[
            pltpu.VMEM((NP,), jnp.float32),
            pltpu.VMEM((NP,), jnp.float32),
            pltpu.VMEM((N_U,), jnp.float32),
            pltpu.VMEM((NP,), jnp.float32),
            pltpu.VMEM((ETT_W,), jnp.int32),
            pltpu.VMEM((ETT_W,), jnp.int32),
            pltpu.VMEM((EU_W,), jnp.int32),
            pltpu.VMEM((EU_W,), jnp.int32),
            pltpu.VMEM((ETT_W,), jnp.float32),
            pltpu.VMEM((EU_W,), jnp.float32),
        ],
    )(_gate_body)
    return f(gsa, gda, gsb, gdb, srcp, dstp, usrcp, udstp)


def _step_body(h_hbm, cu_hbm, alpha_hbm, beta_hbm,
               src_hbm, dst2_hbm, usrc_hbm, udst2_hbm,
               partial_hbm,
               acc, srcg_v, dstg_v, alphag_v, usrc_v, udst2_v, beta_v,
               rows_0, rows_1, rows_2, rows_3,
               sem_0, sem_1, sem_2, sem_3):
    c = lax.axis_index("c")
    s = lax.axis_index("s")
    wid = s * NC + c
    bufs = (rows_0, rows_1, rows_2, rows_3)
    sems = (sem_0, sem_1, sem_2, sem_3)

    zero16 = jnp.zeros((16,), jnp.float32)

    def zrow(i, _):
        for g in range(DH // 16):
            rows_0[i, pl.ds(g * 16, 16)] = zero16
        return _

    lax.fori_loop(0, CH, zrow, None)
    for j in range(ROWS_W // CH):
        pltpu.sync_copy(rows_0, acc.at[pl.ds(s * ROWS_W + j * CH, CH)])

    ubase = wid * EU_W
    pltpu.sync_copy(usrc_hbm.at[pl.ds(ubase, EU_W)], usrc_v)
    pltpu.sync_copy(beta_hbm.at[pl.ds(ubase, EU_W)], beta_v)
    pltpu.sync_copy(udst2_hbm.at[wid], udst2_v)

    plsc.subcore_barrier()

    def scale(buf, w_v, wbase):
        def edge4(m, _):
            for q in range(4):
                e = m * 4 + q
                a = plsc.load_gather(
                    w_v, [jnp.full((16,), wbase + e, jnp.int32)])
                for g in range(DH // 16):
                    buf[e, pl.ds(g * 16, 16)] = buf[e, pl.ds(g * 16, 16)] * a
            return _

        lax.fori_loop(0, CH // 4, edge4, None)

    def run_chunks(table_hbm, idx_v, d2_v, w_v, nch):
        def g_copy(k, b):
            return pltpu.make_async_copy(
                table_hbm.at[idx_v.at[pl.ds(k * CH, CH)]], bufs[b], sems[b])

        def s_copy(k, b):
            return pltpu.make_async_copy(bufs[b], acc.at[d2_v.at[k]], sems[b])

        for b in range(NBUF):
            g_copy(b, b).start()

        def quad(j, _):
            q0 = NBUF * j
            for b in range(NBUF):
                k = q0 + b
                g_copy(k, b).wait()
                scale(bufs[b], w_v, k * CH)
                s_copy(k, b).start(add=True)
                pb = (b - 1) % NBUF
                if b > 0:
                    s_copy(k - 1, pb).wait()

                    @pl.when(k + NBUF - 1 < nch)
                    def _():
                        g_copy(k + NBUF - 1, pb).start()
                else:
                    @pl.when(j > 0)
                    def _():
                        s_copy(k - 1, pb).wait()
                        g_copy(k + NBUF - 1, pb).start()
            return _

        lax.fori_loop(0, nch // NBUF, quad, None)
        s_copy(nch - 1, NBUF - 1).wait()

    def group(g, _):
        base = wid * ETT_W + g * GRP
        pltpu.sync_copy(src_hbm.at[pl.ds(base, GRP)], srcg_v)
        pltpu.sync_copy(alpha_hbm.at[pl.ds(base, GRP)], alphag_v)
        pltpu.sync_copy(dst2_hbm.at[wid, pl.ds(g * GRP_CH, GRP_CH)], dstg_v)
        run_chunks(h_hbm, srcg_v, dstg_v, alphag_v, GRP_CH)
        return _

    lax.fori_loop(0, N_GRP, group, None)
    run_chunks(cu_hbm, usrc_v, udst2_v, beta_v, EU_CH)

    plsc.subcore_barrier()
    for j in range(ROWS_W // CH):
        r = s * ROWS_W + j * CH
        pltpu.sync_copy(acc.at[pl.ds(r, CH)], partial_hbm.at[c, pl.ds(r, CH)])


def _step(h, cu_t, alpha, beta, srcp, dst2, usrcp, udst2):
    mesh = plsc.VectorSubcoreMesh(core_axis_name="c", subcore_axis_name="s")
    f = functools.partial(
        pl.kernel,
        out_type=jax.ShapeDtypeStruct((NC, NP, DH), jnp.float32),
        mesh=mesh,
        compiler_params=pltpu.CompilerParams(needs_layout_passes=False),
        scratch_types=[
            pltpu.VMEM_SHARED((NP, DH), jnp.float32),
            pltpu.VMEM((GRP,), jnp.int32),
            pltpu.VMEM((GRP_CH, CH), jnp.int32),
            pltpu.VMEM((GRP,), jnp.float32),
            pltpu.VMEM((EU_W,), jnp.int32),
            pltpu.VMEM((EU_CH, CH), jnp.int32),
            pltpu.VMEM((EU_W,), jnp.float32),
            pltpu.VMEM((CH, DH), jnp.float32),
            pltpu.VMEM((CH, DH), jnp.float32),
            pltpu.VMEM((CH, DH), jnp.float32),
            pltpu.VMEM((CH, DH), jnp.float32),
            pltpu.SemaphoreType.DMA,
            pltpu.SemaphoreType.DMA,
            pltpu.SemaphoreType.DMA,
            pltpu.SemaphoreType.DMA,
        ],
    )(_step_body)
    return f(h, cu_t, alpha, beta, srcp, dst2, usrcp, udst2)


def _combine_body(p_ref, w_ref, b_ref, h_ref, xs_ref):
    h = p_ref[0] + p_ref[1]
    h_ref[...] = h
    xs_ref[...] = (jnp.sum(h * w_ref[0, :][None, :], axis=1, keepdims=True)
                   + b_ref[0, 0])


def _combine(partial, w_h2x_row, b_h2x):
    return pl.pallas_call(
        _combine_body,
        out_shape=(
            jax.ShapeDtypeStruct((NP, DH), jnp.float32),
            jax.ShapeDtypeStruct((NP, 1), jnp.float32),
        ),
    )(partial, w_h2x_row, b_h2x)


def kernel(pos, pos_u, edge_index_tt, u_src, u_dst, hist_x, history_u, us,
           W_alpha, b_alpha, W_beta, b_beta, W_hist, b_hist,
           W_c2h, b_c2h, W_h2x, b_h2x):
    f32 = jnp.float32
    pos_p = jnp.pad(pos, ((0, NP - N), (0, 0)))
    histx_p = jnp.pad(hist_x, ((0, NP - N), (0, 0)))
    u_full = jnp.concatenate([history_u, us], axis=1)

    src = edge_index_tt[0]
    dst = edge_index_tt[1]
    pad_tt = ETT_P - E_TT
    pad_ids = (jnp.arange(pad_tt, dtype=jnp.int32) % N)
    srcp = jnp.concatenate([src, pad_ids])
    dstp = jnp.concatenate([dst, pad_ids])
    pad_u = EU_P - E_U
    upad_s = (jnp.arange(pad_u, dtype=jnp.int32) % N_U)
    upad_d = (jnp.arange(pad_u, dtype=jnp.int32) % N)
    usrcp = jnp.concatenate([u_src, upad_s])
    udstp = jnp.concatenate([u_dst, upad_d])
    dst2 = dstp.reshape(NW, ETT_CH, CH)
    udst2 = udstp.reshape(NW, EU_CH, CH)

    wa = jnp.concatenate(
        [W_alpha[:, 0], b_alpha, jnp.zeros((1,), f32)]).reshape(1, 8)
    wb = jnp.concatenate(
        [W_beta[:, 0], b_beta, jnp.zeros((1,), f32)]).reshape(1, 8)

    h0, cu, gsa, gda, gsb, gdb = _prep(
        histx_p, W_hist, b_hist.reshape(1, DH), u_full, W_c2h,
        b_c2h.reshape(1, DH), pos_p, wa, pos_u, wb)

    alpha, beta = _gates(
        gsa.reshape(NP), gda.reshape(NP), gsb.reshape(N_U), gdb.reshape(NP),
        srcp, dstp, usrcp, udstp)

    w_row = W_h2x[:, 0].reshape(1, DH)
    b11 = b_h2x.reshape(1, 1)

    h = h0
    xs_cols = []
    for t in range(T):
        partial = _step(h, cu[t], alpha, beta, srcp, dst2, usrcp, udst2)
        h, xs_t = _combine(partial, w_row, b11)
        xs_cols.append(xs_t)

    xs = jnp.concatenate(xs_cols, axis=1)[:N]
    return xs


# speedup vs baseline: 9.8975x; 1.0713x over previous
import functools

import jax
import jax.numpy as jnp
from jax import lax
from jax.experimental import pallas as pl
from jax.experimental.pallas import tpu as pltpu
from jax.experimental.pallas import tpu_sc as plsc

N_G = 8000
N_C = 2000
N = N_G + N_C
NP = 10240
N_U = 2000
E_TT = 320000
E_U = 40000
POS = 3
HIST = 20
UH = 8
T = 4
DH = 128

NC = 2
NS = 16
NW = NC * NS
CH = 64
NBUF = 4

ETT_W = 10240
ETT_P = NW * ETT_W
ETT_CH = ETT_W // CH
EU_W = 1280
EU_P = NW * EU_W
EU_CH = EU_W // CH

ROWS_W = NP // NS
GRP = 1024
GRP_CH = GRP // CH
N_GRP = ETT_W // GRP


def _prep_body(histx_ref, whist_ref, bhist_ref, ufull_ref, wc2h_ref, bc2h_ref,
               pos_ref, wa_ref, posu_ref, wb_ref,
               h0_ref, cu_ref, gsa_ref, gda_ref, gsb_ref, gdb_ref):
    h0_ref[...] = (jnp.dot(histx_ref[...], whist_ref[...],
                           preferred_element_type=jnp.float32)
                   + bhist_ref[0, :][None, :])
    for t in range(T):
        u_win = ufull_ref[:, t:t + UH]
        cu_ref[t] = (jnp.dot(u_win, wc2h_ref[...],
                             preferred_element_type=jnp.float32)
                     + bc2h_ref[0, :][None, :])
    pos = pos_ref[...]
    gsa_ref[...] = (pos[:, 0:1] * wa_ref[0, 0] + pos[:, 1:2] * wa_ref[0, 1]
                    + pos[:, 2:3] * wa_ref[0, 2])
    gda_ref[...] = (pos[:, 0:1] * wa_ref[0, 3] + pos[:, 1:2] * wa_ref[0, 4]
                    + pos[:, 2:3] * wa_ref[0, 5] + wa_ref[0, 6])
    posu = posu_ref[...]
    gsb_ref[...] = (posu[:, 0:1] * wb_ref[0, 0] + posu[:, 1:2] * wb_ref[0, 1]
                    + posu[:, 2:3] * wb_ref[0, 2])
    gdb_ref[...] = (pos[:, 0:1] * wb_ref[0, 3] + pos[:, 1:2] * wb_ref[0, 4]
                    + pos[:, 2:3] * wb_ref[0, 5] + wb_ref[0, 6])


def _prep(histx_p, w_hist, b_hist, u_full, w_c2h, b_c2h, pos_p, wa, pos_u, wb):
    return pl.pallas_call(
        _prep_body,
        out_shape=(
            jax.ShapeDtypeStruct((NP, DH), jnp.float32),
            jax.ShapeDtypeStruct((T, N_U, DH), jnp.float32),
            jax.ShapeDtypeStruct((NP, 1), jnp.float32),
            jax.ShapeDtypeStruct((NP, 1), jnp.float32),
            jax.ShapeDtypeStruct((N_U, 1), jnp.float32),
            jax.ShapeDtypeStruct((NP, 1), jnp.float32),
        ),
    )(histx_p, w_hist, b_hist, u_full, w_c2h, b_c2h, pos_p, wa, pos_u, wb)


def _tanh16(x):
    e = jnp.exp(-2.0 * jnp.abs(x))
    t = (1.0 - e) / (1.0 + e)
    return jnp.where(x < 0.0, -t, t)


def _gate_body(gsa_hbm, gda_hbm, gsb_hbm, gdb_hbm,
               src_hbm, dst_hbm, usrc_hbm, udst_hbm,
               alpha_hbm, beta_hbm,
               gsa_v, gda_v, gsb_v, gdb_v,
               src_v, dst_v, usrc_v, udst_v, alpha_v, beta_v):
    c = lax.axis_index("c")
    s = lax.axis_index("s")
    wid = s * NC + c
    iota = lax.iota(jnp.int32, 16)

    pltpu.sync_copy(gsa_hbm, gsa_v)
    pltpu.sync_copy(gda_hbm, gda_v)
    pltpu.sync_copy(gsb_hbm, gsb_v)
    pltpu.sync_copy(gdb_hbm, gdb_v)

    base = wid * ETT_W
    pltpu.sync_copy(src_hbm.at[pl.ds(base, ETT_W)], src_v)
    pltpu.sync_copy(dst_hbm.at[pl.ds(base, ETT_W)], dst_v)
    ubase = wid * EU_W
    pltpu.sync_copy(usrc_hbm.at[pl.ds(ubase, EU_W)], usrc_v)
    pltpu.sync_copy(udst_hbm.at[pl.ds(ubase, EU_W)], udst_v)

    def tt_step(i, _):
        si = src_v[pl.ds(i * 16, 16)]
        di = dst_v[pl.ds(i * 16, 16)]
        x = (plsc.load_gather(gsa_v, [si]) + plsc.load_gather(gda_v, [di]))
        a = 0.05 * _tanh16(x)
        gidx = base + i * 16 + iota
        alpha_v[pl.ds(i * 16, 16)] = jnp.where(gidx < E_TT, a, 0.0)
        return _

    lax.fori_loop(0, ETT_W // 16, tt_step, None)
    pltpu.sync_copy(alpha_v, alpha_hbm.at[pl.ds(base, ETT_W)])

    def u_step(i, _):
        si = usrc_v[pl.ds(i * 16, 16)]
        di = udst_v[pl.ds(i * 16, 16)]
        x = (plsc.load_gather(gsb_v, [si]) + plsc.load_gather(gdb_v, [di]))
        b = 0.05 * _tanh16(x)
        gidx = ubase + i * 16 + iota
        beta_v[pl.ds(i * 16, 16)] = jnp.where(gidx < E_U, b, 0.0)
        return _

    lax.fori_loop(0, EU_W // 16, u_step, None)
    pltpu.sync_copy(beta_v, beta_hbm.at[pl.ds(ubase, EU_W)])


def _gates(gsa, gda, gsb, gdb, srcp, dstp, usrcp, udstp):
    mesh = plsc.VectorSubcoreMesh(core_axis_name="c", subcore_axis_name="s")
    f = functools.partial(
        pl.kernel,
        out_type=(
            jax.ShapeDtypeStruct((ETT_P,), jnp.float32),
            jax.ShapeDtypeStruct((EU_P,), jnp.float32),
        ),
        mesh=mesh,
        compiler_params=pltpu.CompilerParams(needs_layout_passes=False),
        scratch_types=[
            pltpu.VMEM((NP,), jnp.float32),
            pltpu.VMEM((NP,), jnp.float32),
            pltpu.VMEM((N_U,), jnp.float32),
            pltpu.VMEM((NP,), jnp.float32),
            pltpu.VMEM((ETT_W,), jnp.int32),
            pltpu.VMEM((ETT_W,), jnp.int32),
            pltpu.VMEM((EU_W,), jnp.int32),
            pltpu.VMEM((EU_W,), jnp.int32),
            pltpu.VMEM((ETT_W,), jnp.float32),
            pltpu.VMEM((EU_W,), jnp.float32),
        ],
    )(_gate_body)
    return f(gsa, gda, gsb, gdb, srcp, dstp, usrcp, udstp)


def _step_body(h_hbm, cu_hbm, alpha_hbm, beta_hbm,
               src_hbm, dst2_hbm, usrc_hbm, udst2_hbm,
               partial_hbm,
               acc, srcg_0, dstg_0, alphag_0, srcg_1, dstg_1, alphag_1,
               usrc_v, udst2_v, beta_v,
               rows_0, rows_1, rows_2, rows_3,
               sem_0, sem_1, sem_2, sem_3, sem_st):
    c = lax.axis_index("c")
    s = lax.axis_index("s")
    wid = s * NC + c
    bufs = (rows_0, rows_1, rows_2, rows_3)
    sems = (sem_0, sem_1, sem_2, sem_3)
    stage_bufs = ((srcg_0, dstg_0, alphag_0), (srcg_1, dstg_1, alphag_1))

    zero16 = jnp.zeros((16,), jnp.float32)

    def zrow(i, _):
        for g in range(DH // 16):
            rows_0[i, pl.ds(g * 16, 16)] = zero16
        return _

    lax.fori_loop(0, CH, zrow, None)
    for j in range(ROWS_W // CH):
        pltpu.sync_copy(rows_0, acc.at[pl.ds(s * ROWS_W + j * CH, CH)])

    ubase = wid * EU_W
    pltpu.sync_copy(usrc_hbm.at[pl.ds(ubase, EU_W)], usrc_v)
    pltpu.sync_copy(beta_hbm.at[pl.ds(ubase, EU_W)], beta_v)
    pltpu.sync_copy(udst2_hbm.at[wid], udst2_v)

    plsc.subcore_barrier()

    def scale(buf, w_v, wbase):
        def edge8(m, _):
            for q in range(8):
                e = m * 8 + q
                a = plsc.load_gather(
                    w_v, [jnp.full((16,), wbase + e, jnp.int32)])
                for g in range(DH // 16):
                    buf[e, pl.ds(g * 16, 16)] = buf[e, pl.ds(g * 16, 16)] * a
            return _

        lax.fori_loop(0, CH // 8, edge8, None)

    def run_chunks(table_hbm, idx_v, d2_v, w_v, nch):
        def g_copy(k, b):
            return pltpu.make_async_copy(
                table_hbm.at[idx_v.at[pl.ds(k * CH, CH)]], bufs[b], sems[b])

        def s_copy(k, b):
            return pltpu.make_async_copy(bufs[b], acc.at[d2_v.at[k]], sems[b])

        for b in range(NBUF):
            g_copy(b, b).start()

        def quad(j, _):
            q0 = NBUF * j
            for b in range(NBUF):
                k = q0 + b
                g_copy(k, b).wait()
                scale(bufs[b], w_v, k * CH)
                s_copy(k, b).start(add=True)
                pb = (b - 1) % NBUF
                if b > 0:
                    s_copy(k - 1, pb).wait()

                    @pl.when(k + NBUF - 1 < nch)
                    def _():
                        g_copy(k + NBUF - 1, pb).start()
                else:
                    @pl.when(j > 0)
                    def _():
                        s_copy(k - 1, pb).wait()
                        g_copy(k + NBUF - 1, pb).start()
            return _

        lax.fori_loop(0, nch // NBUF, quad, None)
        s_copy(nch - 1, NBUF - 1).wait()

    def stage_copies(g, bset):
        sg, dg, ag = bset
        base = wid * ETT_W + g * GRP
        return (
            pltpu.make_async_copy(src_hbm.at[pl.ds(base, GRP)], sg, sem_st),
            pltpu.make_async_copy(alpha_hbm.at[pl.ds(base, GRP)], ag, sem_st),
            pltpu.make_async_copy(
                dst2_hbm.at[wid, pl.ds(g * GRP_CH, GRP_CH)], dg, sem_st),
        )

    def stage_start(g, bset):
        for cp in stage_copies(g, bset):
            cp.start()

    def stage_wait(g, bset):
        for cp in stage_copies(g, bset):
            cp.wait()

    stage_start(0, stage_bufs[0])
    stage_wait(0, stage_bufs[0])

    def gpair(j, _):
        g0 = 2 * j
        stage_start(g0 + 1, stage_bufs[1])
        sg, dg, ag = stage_bufs[0]
        run_chunks(h_hbm, sg, dg, ag, GRP_CH)
        stage_wait(g0 + 1, stage_bufs[1])

        @pl.when(g0 + 2 < N_GRP)
        def _():
            stage_start(g0 + 2, stage_bufs[0])

        sg, dg, ag = stage_bufs[1]
        run_chunks(h_hbm, sg, dg, ag, GRP_CH)

        @pl.when(g0 + 2 < N_GRP)
        def _():
            stage_wait(g0 + 2, stage_bufs[0])
        return _

    lax.fori_loop(0, N_GRP // 2, gpair, None)
    run_chunks(cu_hbm, usrc_v, udst2_v, beta_v, EU_CH)

    plsc.subcore_barrier()
    for j in range(ROWS_W // CH):
        r = s * ROWS_W + j * CH
        pltpu.sync_copy(acc.at[pl.ds(r, CH)], partial_hbm.at[c, pl.ds(r, CH)])


def _step(h, cu_t, alpha, beta, srcp, dst2, usrcp, udst2):
    mesh = plsc.VectorSubcoreMesh(core_axis_name="c", subcore_axis_name="s")
    f = functools.partial(
        pl.kernel,
        out_type=jax.ShapeDtypeStruct((NC, NP, DH), jnp.float32),
        mesh=mesh,
        compiler_params=pltpu.CompilerParams(needs_layout_passes=False),
        scratch_types=[
            pltpu.VMEM_SHARED((NP, DH), jnp.float32),
            pltpu.VMEM((GRP,), jnp.int32),
            pltpu.VMEM((GRP_CH, CH), jnp.int32),
            pltpu.VMEM((GRP,), jnp.float32),
            pltpu.VMEM((GRP,), jnp.int32),
            pltpu.VMEM((GRP_CH, CH), jnp.int32),
            pltpu.VMEM((GRP,), jnp.float32),
            pltpu.VMEM((EU_W,), jnp.int32),
            pltpu.VMEM((EU_CH, CH), jnp.int32),
            pltpu.VMEM((EU_W,), jnp.float32),
            pltpu.VMEM((CH, DH), jnp.float32),
            pltpu.VMEM((CH, DH), jnp.float32),
            pltpu.VMEM((CH, DH), jnp.float32),
            pltpu.VMEM((CH, DH), jnp.float32),
            pltpu.SemaphoreType.DMA,
            pltpu.SemaphoreType.DMA,
            pltpu.SemaphoreType.DMA,
            pltpu.SemaphoreType.DMA,
            pltpu.SemaphoreType.DMA,
        ],
    )(_step_body)
    return f(h, cu_t, alpha, beta, srcp, dst2, usrcp, udst2)


def _combine_body(p_ref, w_ref, b_ref, h_ref, xs_ref):
    h = p_ref[0] + p_ref[1]
    h_ref[...] = h
    xs_ref[...] = (jnp.sum(h * w_ref[0, :][None, :], axis=1, keepdims=True)
                   + b_ref[0, 0])


def _combine(partial, w_h2x_row, b_h2x):
    return pl.pallas_call(
        _combine_body,
        out_shape=(
            jax.ShapeDtypeStruct((NP, DH), jnp.float32),
            jax.ShapeDtypeStruct((NP, 1), jnp.float32),
        ),
    )(partial, w_h2x_row, b_h2x)


def kernel(pos, pos_u, edge_index_tt, u_src, u_dst, hist_x, history_u, us,
           W_alpha, b_alpha, W_beta, b_beta, W_hist, b_hist,
           W_c2h, b_c2h, W_h2x, b_h2x):
    f32 = jnp.float32
    pos_p = jnp.pad(pos, ((0, NP - N), (0, 0)))
    histx_p = jnp.pad(hist_x, ((0, NP - N), (0, 0)))
    u_full = jnp.concatenate([history_u, us], axis=1)

    src = edge_index_tt[0]
    dst = edge_index_tt[1]
    pad_tt = ETT_P - E_TT
    pad_ids = (jnp.arange(pad_tt, dtype=jnp.int32) % N)
    srcp = jnp.concatenate([src, pad_ids])
    dstp = jnp.concatenate([dst, pad_ids])
    pad_u = EU_P - E_U
    upad_s = (jnp.arange(pad_u, dtype=jnp.int32) % N_U)
    upad_d = (jnp.arange(pad_u, dtype=jnp.int32) % N)
    usrcp = jnp.concatenate([u_src, upad_s])
    udstp = jnp.concatenate([u_dst, upad_d])
    dst2 = dstp.reshape(NW, ETT_CH, CH)
    udst2 = udstp.reshape(NW, EU_CH, CH)

    wa = jnp.concatenate(
        [W_alpha[:, 0], b_alpha, jnp.zeros((1,), f32)]).reshape(1, 8)
    wb = jnp.concatenate(
        [W_beta[:, 0], b_beta, jnp.zeros((1,), f32)]).reshape(1, 8)

    h0, cu, gsa, gda, gsb, gdb = _prep(
        histx_p, W_hist, b_hist.reshape(1, DH), u_full, W_c2h,
        b_c2h.reshape(1, DH), pos_p, wa, pos_u, wb)

    alpha, beta = _gates(
        gsa.reshape(NP), gda.reshape(NP), gsb.reshape(N_U), gdb.reshape(NP),
        srcp, dstp, usrcp, udstp)

    w_row = W_h2x[:, 0].reshape(1, DH)
    b11 = b_h2x.reshape(1, 1)

    h = h0
    xs_cols = []
    for t in range(T):
        partial = _step(h, cu[t], alpha, beta, srcp, dst2, usrcp, udst2)
        h, xs_t = _combine(partial, w_row, b11)
        xs_cols.append(xs_t)

    xs = jnp.concatenate(xs_cols, axis=1)[:N]
    return xs


# speedup vs baseline: 10.2534x; 1.0360x over previous
import functools

import jax
import jax.numpy as jnp
from jax import lax
from jax.experimental import pallas as pl
from jax.experimental.pallas import tpu as pltpu
from jax.experimental.pallas import tpu_sc as plsc

N_G = 8000
N_C = 2000
N = N_G + N_C
NP = 10240
N_U = 2000
E_TT = 320000
E_U = 40000
POS = 3
HIST = 20
UH = 8
T = 4
DH = 128

NC = 2
NS = 16
NW = NC * NS
CH = 64
NBUF = 4

ETT_W = 10240
ETT_P = NW * ETT_W
ETT_CH = ETT_W // CH
EU_W = 1280
EU_P = NW * EU_W
EU_CH = EU_W // CH

ROWS_W = NP // NS
GRP = 1024
GRP_CH = GRP // CH
N_GRP = ETT_W // GRP


def _prep_body(histx_ref, whist_ref, bhist_ref, ufull_ref, wc2h_ref, bc2h_ref,
               h0_ref, cu_ref):
    h0_ref[...] = (jnp.dot(histx_ref[...], whist_ref[...],
                           preferred_element_type=jnp.float32)
                   + bhist_ref[0, :][None, :])
    for t in range(T):
        u_win = ufull_ref[:, t:t + UH]
        cu_ref[t] = (jnp.dot(u_win, wc2h_ref[...],
                             preferred_element_type=jnp.float32)
                     + bc2h_ref[0, :][None, :])


def _prep(histx_p, w_hist, b_hist, u_full, w_c2h, b_c2h):
    return pl.pallas_call(
        _prep_body,
        out_shape=(
            jax.ShapeDtypeStruct((NP, DH), jnp.float32),
            jax.ShapeDtypeStruct((T, N_U, DH), jnp.float32),
        ),
    )(histx_p, w_hist, b_hist, u_full, w_c2h, b_c2h)


def _tanh16(x):
    e = jnp.exp(-2.0 * jnp.abs(x))
    t = (1.0 - e) / (1.0 + e)
    return jnp.where(x < 0.0, -t, t)


def _gate_body(posT_hbm, posuT_hbm, w_hbm,
               src_hbm, dst_hbm, usrc_hbm, udst_hbm,
               alpha_hbm, beta_hbm,
               gsa_v, gda_v, gsb_v, gdb_v,
               posT_v, posuT_v, w_v,
               src_v, dst_v, usrc_v, udst_v, alpha_v, beta_v):
    c = lax.axis_index("c")
    s = lax.axis_index("s")
    wid = s * NC + c
    iota = lax.iota(jnp.int32, 16)

    pltpu.sync_copy(posT_hbm, posT_v)
    pltpu.sync_copy(posuT_hbm, posuT_v)
    pltpu.sync_copy(w_hbm, w_v)

    def wscal(r, i):
        return plsc.load_gather(w_v, [jnp.full((16,), r * 8 + i, jnp.int32)])

    def node_tab(i, _):
        sl = pl.ds(i * 16, 16)
        p0 = posT_v[0, sl]
        p1 = posT_v[1, sl]
        p2 = posT_v[2, sl]
        gsa_v[sl] = (p0 * wscal(0, 0) + p1 * wscal(0, 1) + p2 * wscal(0, 2))
        gda_v[sl] = (p0 * wscal(0, 3) + p1 * wscal(0, 4) + p2 * wscal(0, 5)
                     + wscal(0, 6))
        gdb_v[sl] = (p0 * wscal(1, 3) + p1 * wscal(1, 4) + p2 * wscal(1, 5)
                     + wscal(1, 6))
        return _

    lax.fori_loop(0, NP // 16, node_tab, None)

    def unode_tab(i, _):
        sl = pl.ds(i * 16, 16)
        gsb_v[sl] = (posuT_v[0, sl] * wscal(1, 0) + posuT_v[1, sl] * wscal(1, 1)
                     + posuT_v[2, sl] * wscal(1, 2))
        return _

    lax.fori_loop(0, N_U // 16, unode_tab, None)

    base = wid * ETT_W
    pltpu.sync_copy(src_hbm.at[pl.ds(base, ETT_W)], src_v)
    pltpu.sync_copy(dst_hbm.at[pl.ds(base, ETT_W)], dst_v)
    ubase = wid * EU_W
    pltpu.sync_copy(usrc_hbm.at[pl.ds(ubase, EU_W)], usrc_v)
    pltpu.sync_copy(udst_hbm.at[pl.ds(ubase, EU_W)], udst_v)

    def tt_step(i, _):
        si = src_v[pl.ds(i * 16, 16)]
        di = dst_v[pl.ds(i * 16, 16)]
        x = (plsc.load_gather(gsa_v, [si]) + plsc.load_gather(gda_v, [di]))
        a = 0.05 * _tanh16(x)
        gidx = base + i * 16 + iota
        alpha_v[pl.ds(i * 16, 16)] = jnp.where(gidx < E_TT, a, 0.0)
        return _

    lax.fori_loop(0, ETT_W // 16, tt_step, None)
    pltpu.sync_copy(alpha_v, alpha_hbm.at[pl.ds(base, ETT_W)])

    def u_step(i, _):
        si = usrc_v[pl.ds(i * 16, 16)]
        di = udst_v[pl.ds(i * 16, 16)]
        x = (plsc.load_gather(gsb_v, [si]) + plsc.load_gather(gdb_v, [di]))
        b = 0.05 * _tanh16(x)
        gidx = ubase + i * 16 + iota
        beta_v[pl.ds(i * 16, 16)] = jnp.where(gidx < E_U, b, 0.0)
        return _

    lax.fori_loop(0, EU_W // 16, u_step, None)
    pltpu.sync_copy(beta_v, beta_hbm.at[pl.ds(ubase, EU_W)])


def _gates(posT, posuT, w2, srcp, dstp, usrcp, udstp):
    mesh = plsc.VectorSubcoreMesh(core_axis_name="c", subcore_axis_name="s")
    f = functools.partial(
        pl.kernel,
        out_type=(
            jax.ShapeDtypeStruct((ETT_P,), jnp.float32),
            jax.ShapeDtypeStruct((EU_P,), jnp.float32),
        ),
        mesh=mesh,
        compiler_params=pltpu.CompilerParams(needs_layout_passes=False),
        scratch_types=[
            pltpu.VMEM((NP,), jnp.float32),
            pltpu.VMEM((NP,), jnp.float32),
            pltpu.VMEM((N_U,), jnp.float32),
            pltpu.VMEM((NP,), jnp.float32),
            pltpu.VMEM((POS, NP), jnp.float32),
            pltpu.VMEM((POS, N_U), jnp.float32),
            pltpu.VMEM((16,), jnp.float32),
            pltpu.VMEM((ETT_W,), jnp.int32),
            pltpu.VMEM((ETT_W,), jnp.int32),
            pltpu.VMEM((EU_W,), jnp.int32),
            pltpu.VMEM((EU_W,), jnp.int32),
            pltpu.VMEM((ETT_W,), jnp.float32),
            pltpu.VMEM((EU_W,), jnp.float32),
        ],
    )(_gate_body)
    return f(posT, posuT, w2, srcp, dstp, usrcp, udstp)


def _step_body(h_hbm, cu_hbm, alpha_hbm, beta_hbm,
               src_hbm, dst2_hbm, usrc_hbm, udst2_hbm,
               partial_hbm,
               acc, srcg_0, dstg_0, alphag_0, srcg_1, dstg_1, alphag_1,
               usrc_v, udst2_v, beta_v,
               rows_0, rows_1, rows_2, rows_3,
               sem_0, sem_1, sem_2, sem_3, sem_st):
    c = lax.axis_index("c")
    s = lax.axis_index("s")
    wid = s * NC + c
    bufs = (rows_0, rows_1, rows_2, rows_3)
    sems = (sem_0, sem_1, sem_2, sem_3)
    stage_bufs = ((srcg_0, dstg_0, alphag_0), (srcg_1, dstg_1, alphag_1))

    zero16 = jnp.zeros((16,), jnp.float32)

    def zrow(i, _):
        for g in range(DH // 16):
            rows_0[i, pl.ds(g * 16, 16)] = zero16
        return _

    lax.fori_loop(0, CH, zrow, None)
    zcps = [pltpu.make_async_copy(
        rows_0, acc.at[pl.ds(s * ROWS_W + j * CH, CH)], sem_st)
        for j in range(ROWS_W // CH)]
    for cp in zcps:
        cp.start()
    for cp in zcps:
        cp.wait()

    ubase = wid * EU_W
    pltpu.sync_copy(usrc_hbm.at[pl.ds(ubase, EU_W)], usrc_v)
    pltpu.sync_copy(beta_hbm.at[pl.ds(ubase, EU_W)], beta_v)
    pltpu.sync_copy(udst2_hbm.at[wid], udst2_v)

    plsc.subcore_barrier()

    def scale(buf, w_v, wbase):
        def edge8(m, _):
            for q in range(8):
                e = m * 8 + q
                a = plsc.load_gather(
                    w_v, [jnp.full((16,), wbase + e, jnp.int32)])
                for g in range(DH // 16):
                    buf[e, pl.ds(g * 16, 16)] = buf[e, pl.ds(g * 16, 16)] * a
            return _

        lax.fori_loop(0, CH // 8, edge8, None)

    def run_chunks(table_hbm, idx_v, d2_v, w_v, nch):
        def g_copy(k, b):
            return pltpu.make_async_copy(
                table_hbm.at[idx_v.at[pl.ds(k * CH, CH)]], bufs[b], sems[b])

        def s_copy(k, b):
            return pltpu.make_async_copy(bufs[b], acc.at[d2_v.at[k]], sems[b])

        for b in range(NBUF):
            g_copy(b, b).start()

        def quad(j, _):
            q0 = NBUF * j
            for b in range(NBUF):
                k = q0 + b
                g_copy(k, b).wait()
                scale(bufs[b], w_v, k * CH)
                s_copy(k, b).start(add=True)
                pb = (b - 1) % NBUF
                if b > 0:
                    s_copy(k - 1, pb).wait()

                    @pl.when(k + NBUF - 1 < nch)
                    def _():
                        g_copy(k + NBUF - 1, pb).start()
                else:
                    @pl.when(j > 0)
                    def _():
                        s_copy(k - 1, pb).wait()
                        g_copy(k + NBUF - 1, pb).start()
            return _

        lax.fori_loop(0, nch // NBUF, quad, None)
        s_copy(nch - 1, NBUF - 1).wait()

    def stage_copies(g, bset):
        sg, dg, ag = bset
        base = wid * ETT_W + g * GRP
        return (
            pltpu.make_async_copy(src_hbm.at[pl.ds(base, GRP)], sg, sem_st),
            pltpu.make_async_copy(alpha_hbm.at[pl.ds(base, GRP)], ag, sem_st),
            pltpu.make_async_copy(
                dst2_hbm.at[wid, pl.ds(g * GRP_CH, GRP_CH)], dg, sem_st),
        )

    def stage_start(g, bset):
        for cp in stage_copies(g, bset):
            cp.start()

    def stage_wait(g, bset):
        for cp in stage_copies(g, bset):
            cp.wait()

    stage_start(0, stage_bufs[0])
    stage_wait(0, stage_bufs[0])

    def gpair(j, _):
        g0 = 2 * j
        stage_start(g0 + 1, stage_bufs[1])
        sg, dg, ag = stage_bufs[0]
        run_chunks(h_hbm, sg, dg, ag, GRP_CH)
        stage_wait(g0 + 1, stage_bufs[1])

        @pl.when(g0 + 2 < N_GRP)
        def _():
            stage_start(g0 + 2, stage_bufs[0])

        sg, dg, ag = stage_bufs[1]
        run_chunks(h_hbm, sg, dg, ag, GRP_CH)

        @pl.when(g0 + 2 < N_GRP)
        def _():
            stage_wait(g0 + 2, stage_bufs[0])
        return _

    lax.fori_loop(0, N_GRP // 2, gpair, None)
    run_chunks(cu_hbm, usrc_v, udst2_v, beta_v, EU_CH)

    plsc.subcore_barrier()
    ocps = [pltpu.make_async_copy(
        acc.at[pl.ds(s * ROWS_W + j * CH, CH)],
        partial_hbm.at[c, pl.ds(s * ROWS_W + j * CH, CH)], sem_st)
        for j in range(ROWS_W // CH)]
    for cp in ocps:
        cp.start()
    for cp in ocps:
        cp.wait()


def _step(h, cu_t, alpha, beta, srcp, dst2, usrcp, udst2):
    mesh = plsc.VectorSubcoreMesh(core_axis_name="c", subcore_axis_name="s")
    f = functools.partial(
        pl.kernel,
        out_type=jax.ShapeDtypeStruct((NC, NP, DH), jnp.float32),
        mesh=mesh,
        compiler_params=pltpu.CompilerParams(needs_layout_passes=False),
        scratch_types=[
            pltpu.VMEM_SHARED((NP, DH), jnp.float32),
            pltpu.VMEM((GRP,), jnp.int32),
            pltpu.VMEM((GRP_CH, CH), jnp.int32),
            pltpu.VMEM((GRP,), jnp.float32),
            pltpu.VMEM((GRP,), jnp.int32),
            pltpu.VMEM((GRP_CH, CH), jnp.int32),
            pltpu.VMEM((GRP,), jnp.float32),
            pltpu.VMEM((EU_W,), jnp.int32),
            pltpu.VMEM((EU_CH, CH), jnp.int32),
            pltpu.VMEM((EU_W,), jnp.float32),
            pltpu.VMEM((CH, DH), jnp.float32),
            pltpu.VMEM((CH, DH), jnp.float32),
            pltpu.VMEM((CH, DH), jnp.float32),
            pltpu.VMEM((CH, DH), jnp.float32),
            pltpu.SemaphoreType.DMA,
            pltpu.SemaphoreType.DMA,
            pltpu.SemaphoreType.DMA,
            pltpu.SemaphoreType.DMA,
            pltpu.SemaphoreType.DMA,
        ],
    )(_step_body)
    return f(h, cu_t, alpha, beta, srcp, dst2, usrcp, udst2)


def _combine_body(p_ref, w_ref, b_ref, h_ref, xs_ref):
    h = p_ref[0] + p_ref[1]
    h_ref[...] = h
    xs_ref[...] = (jnp.sum(h * w_ref[0, :][None, :], axis=1, keepdims=True)
                   + b_ref[0, 0])


def _combine(partial, w_h2x_row, b_h2x):
    return pl.pallas_call(
        _combine_body,
        out_shape=(
            jax.ShapeDtypeStruct((NP, DH), jnp.float32),
            jax.ShapeDtypeStruct((NP, 1), jnp.float32),
        ),
    )(partial, w_h2x_row, b_h2x)


def kernel(pos, pos_u, edge_index_tt, u_src, u_dst, hist_x, history_u, us,
           W_alpha, b_alpha, W_beta, b_beta, W_hist, b_hist,
           W_c2h, b_c2h, W_h2x, b_h2x):
    f32 = jnp.float32
    pos_p = jnp.pad(pos, ((0, NP - N), (0, 0)))
    histx_p = jnp.pad(hist_x, ((0, NP - N), (0, 0)))
    u_full = jnp.concatenate([history_u, us], axis=1)

    src = edge_index_tt[0]
    dst = edge_index_tt[1]
    pad_tt = ETT_P - E_TT
    pad_ids = (jnp.arange(pad_tt, dtype=jnp.int32) % N)
    srcp = jnp.concatenate([src, pad_ids])
    dstp = jnp.concatenate([dst, pad_ids])
    pad_u = EU_P - E_U
    upad_s = (jnp.arange(pad_u, dtype=jnp.int32) % N_U)
    upad_d = (jnp.arange(pad_u, dtype=jnp.int32) % N)
    usrcp = jnp.concatenate([u_src, upad_s])
    udstp = jnp.concatenate([u_dst, upad_d])
    dst2 = dstp.reshape(NW, ETT_CH, CH)
    udst2 = udstp.reshape(NW, EU_CH, CH)

    w2 = jnp.concatenate(
        [W_alpha[:, 0], b_alpha, jnp.zeros((1,), f32),
         W_beta[:, 0], b_beta, jnp.zeros((1,), f32)])
    posT = pos_p.T.reshape(POS, NP)
    posuT = pos_u.T.reshape(POS, N_U)

    h0, cu = _prep(histx_p, W_hist, b_hist.reshape(1, DH), u_full, W_c2h,
                   b_c2h.reshape(1, DH))

    alpha, beta = _gates(posT, posuT, w2, srcp, dstp, usrcp, udstp)

    w_row = W_h2x[:, 0].reshape(1, DH)
    b11 = b_h2x.reshape(1, 1)

    h = h0
    xs_cols = []
    for t in range(T):
        partial = _step(h, cu[t], alpha, beta, srcp, dst2, usrcp, udst2)
        h, xs_t = _combine(partial, w_row, b11)
        xs_cols.append(xs_t)

    xs = jnp.concatenate(xs_cols, axis=1)[:N]
    return xs


# speedup vs baseline: 10.3116x; 1.0057x over previous
import functools

import jax
import jax.numpy as jnp
from jax import lax
from jax.experimental import pallas as pl
from jax.experimental.pallas import tpu as pltpu
from jax.experimental.pallas import tpu_sc as plsc

N_G = 8000
N_C = 2000
N = N_G + N_C
NP = 10240
N_U = 2000
E_TT = 320000
E_U = 40000
POS = 3
HIST = 20
UH = 8
T = 4
DH = 128

NC = 2
NS = 16
NW = NC * NS
CH = 64
NBUF = 4

ETT_W = 10240
ETT_P = NW * ETT_W
ETT_CH = ETT_W // CH
EU_W = 1280
EU_P = NW * EU_W
EU_CH = EU_W // CH

ROWS_W = NP // NS
GRP = 1024
GRP_CH = GRP // CH
N_GRP = ETT_W // GRP


def _prep_body(histx_ref, whist_ref, bhist_ref, ufull_ref, wc2h_ref, bc2h_ref,
               h0_ref, cu_ref):
    h0_ref[...] = (jnp.dot(histx_ref[...], whist_ref[...],
                           preferred_element_type=jnp.float32)
                   + bhist_ref[0, :][None, :])
    for t in range(T):
        u_win = ufull_ref[:, t:t + UH]
        cu_ref[t] = (jnp.dot(u_win, wc2h_ref[...],
                             preferred_element_type=jnp.float32)
                     + bc2h_ref[0, :][None, :])


def _prep(histx_p, w_hist, b_hist, u_full, w_c2h, b_c2h):
    return pl.pallas_call(
        _prep_body,
        out_shape=(
            jax.ShapeDtypeStruct((NP, DH), jnp.float32),
            jax.ShapeDtypeStruct((T, N_U, DH), jnp.float32),
        ),
    )(histx_p, w_hist, b_hist, u_full, w_c2h, b_c2h)


def _tanh16(x):
    e = jnp.exp(-2.0 * jnp.abs(x))
    t = (1.0 - e) / (1.0 + e)
    return jnp.where(x < 0.0, -t, t)


def _gate_body(p0_hbm, p1_hbm, p2_hbm, q0_hbm, q1_hbm, q2_hbm, w_hbm,
               src_hbm, dst_hbm, usrc_hbm, udst_hbm,
               alpha_hbm, beta_hbm,
               gsa_v, gda_v, gsb_v, gdb_v,
               p0_v, p1_v, p2_v, q0_v, q1_v, q2_v, w_v,
               src_v, dst_v, usrc_v, udst_v, alpha_v, beta_v):
    c = lax.axis_index("c")
    s = lax.axis_index("s")
    wid = s * NC + c
    iota = lax.iota(jnp.int32, 16)

    pltpu.sync_copy(p0_hbm, p0_v)
    pltpu.sync_copy(p1_hbm, p1_v)
    pltpu.sync_copy(p2_hbm, p2_v)
    pltpu.sync_copy(q0_hbm, q0_v)
    pltpu.sync_copy(q1_hbm, q1_v)
    pltpu.sync_copy(q2_hbm, q2_v)
    pltpu.sync_copy(w_hbm, w_v)

    def wscal(r, i):
        return w_v[pl.ds((r * 8 + i) * 16, 16)]

    def node_tab(i, _):
        sl = pl.ds(i * 16, 16)
        p0 = p0_v[sl]
        p1 = p1_v[sl]
        p2 = p2_v[sl]
        gsa_v[sl] = (p0 * wscal(0, 0) + p1 * wscal(0, 1) + p2 * wscal(0, 2))
        gda_v[sl] = (p0 * wscal(0, 3) + p1 * wscal(0, 4) + p2 * wscal(0, 5)
                     + wscal(0, 6))
        gdb_v[sl] = (p0 * wscal(1, 3) + p1 * wscal(1, 4) + p2 * wscal(1, 5)
                     + wscal(1, 6))
        return _

    lax.fori_loop(0, NP // 16, node_tab, None)

    def unode_tab(i, _):
        sl = pl.ds(i * 16, 16)
        gsb_v[sl] = (q0_v[sl] * wscal(1, 0) + q1_v[sl] * wscal(1, 1)
                     + q2_v[sl] * wscal(1, 2))
        return _

    lax.fori_loop(0, N_U // 16, unode_tab, None)

    base = wid * ETT_W
    pltpu.sync_copy(src_hbm.at[pl.ds(base, ETT_W)], src_v)
    pltpu.sync_copy(dst_hbm.at[pl.ds(base, ETT_W)], dst_v)
    ubase = wid * EU_W
    pltpu.sync_copy(usrc_hbm.at[pl.ds(ubase, EU_W)], usrc_v)
    pltpu.sync_copy(udst_hbm.at[pl.ds(ubase, EU_W)], udst_v)

    def tt_step(i, _):
        si = src_v[pl.ds(i * 16, 16)]
        di = dst_v[pl.ds(i * 16, 16)]
        x = (plsc.load_gather(gsa_v, [si]) + plsc.load_gather(gda_v, [di]))
        a = 0.05 * _tanh16(x)
        gidx = base + i * 16 + iota
        alpha_v[pl.ds(i * 16, 16)] = jnp.where(gidx < E_TT, a, 0.0)
        return _

    lax.fori_loop(0, ETT_W // 16, tt_step, None)
    pltpu.sync_copy(alpha_v, alpha_hbm.at[pl.ds(base, ETT_W)])

    def u_step(i, _):
        si = usrc_v[pl.ds(i * 16, 16)]
        di = udst_v[pl.ds(i * 16, 16)]
        x = (plsc.load_gather(gsb_v, [si]) + plsc.load_gather(gdb_v, [di]))
        b = 0.05 * _tanh16(x)
        gidx = ubase + i * 16 + iota
        beta_v[pl.ds(i * 16, 16)] = jnp.where(gidx < E_U, b, 0.0)
        return _

    lax.fori_loop(0, EU_W // 16, u_step, None)
    pltpu.sync_copy(beta_v, beta_hbm.at[pl.ds(ubase, EU_W)])


def _gates(pcols, qcols, w2, srcp, dstp, usrcp, udstp):
    mesh = plsc.VectorSubcoreMesh(core_axis_name="c", subcore_axis_name="s")
    f = functools.partial(
        pl.kernel,
        out_type=(
            jax.ShapeDtypeStruct((ETT_P,), jnp.float32),
            jax.ShapeDtypeStruct((EU_P,), jnp.float32),
        ),
        mesh=mesh,
        compiler_params=pltpu.CompilerParams(needs_layout_passes=False),
        scratch_types=[
            pltpu.VMEM((NP,), jnp.float32),
            pltpu.VMEM((NP,), jnp.float32),
            pltpu.VMEM((N_U,), jnp.float32),
            pltpu.VMEM((NP,), jnp.float32),
            pltpu.VMEM((NP,), jnp.float32),
            pltpu.VMEM((NP,), jnp.float32),
            pltpu.VMEM((NP,), jnp.float32),
            pltpu.VMEM((N_U,), jnp.float32),
            pltpu.VMEM((N_U,), jnp.float32),
            pltpu.VMEM((N_U,), jnp.float32),
            pltpu.VMEM((256,), jnp.float32),
            pltpu.VMEM((ETT_W,), jnp.int32),
            pltpu.VMEM((ETT_W,), jnp.int32),
            pltpu.VMEM((EU_W,), jnp.int32),
            pltpu.VMEM((EU_W,), jnp.int32),
            pltpu.VMEM((ETT_W,), jnp.float32),
            pltpu.VMEM((EU_W,), jnp.float32),
        ],
    )(_gate_body)
    return f(*pcols, *qcols, w2, srcp, dstp, usrcp, udstp)


def _step_body(h_hbm, cu_hbm, alpha_hbm, beta_hbm,
               src_hbm, dst2_hbm, usrc_hbm, udst2_hbm,
               partial_hbm,
               acc, srcg_0, dstg_0, alphag_0, srcg_1, dstg_1, alphag_1,
               usrc_v, udst2_v, beta_v,
               rows_0, rows_1, rows_2, rows_3,
               sem_0, sem_1, sem_2, sem_3, sem_st):
    c = lax.axis_index("c")
    s = lax.axis_index("s")
    wid = s * NC + c
    bufs = (rows_0, rows_1, rows_2, rows_3)
    sems = (sem_0, sem_1, sem_2, sem_3)
    stage_bufs = ((srcg_0, dstg_0, alphag_0), (srcg_1, dstg_1, alphag_1))

    zero16 = jnp.zeros((16,), jnp.float32)

    def zrow(i, _):
        for g in range(DH // 16):
            rows_0[i, pl.ds(g * 16, 16)] = zero16
        return _

    lax.fori_loop(0, CH, zrow, None)
    zcps = [pltpu.make_async_copy(
        rows_0, acc.at[pl.ds(s * ROWS_W + j * CH, CH)], sem_st)
        for j in range(ROWS_W // CH)]
    for cp in zcps:
        cp.start()
    for cp in zcps:
        cp.wait()

    ubase = wid * EU_W
    pltpu.sync_copy(usrc_hbm.at[pl.ds(ubase, EU_W)], usrc_v)
    pltpu.sync_copy(beta_hbm.at[pl.ds(ubase, EU_W)], beta_v)
    pltpu.sync_copy(udst2_hbm.at[wid], udst2_v)

    plsc.subcore_barrier()

    def scale(buf, w_v, wbase):
        def edge8(m, _):
            for q in range(8):
                e = m * 8 + q
                a = plsc.load_gather(
                    w_v, [jnp.full((16,), wbase + e, jnp.int32)])
                for g in range(DH // 16):
                    buf[e, pl.ds(g * 16, 16)] = buf[e, pl.ds(g * 16, 16)] * a
            return _

        lax.fori_loop(0, CH // 8, edge8, None)

    def run_chunks(table_hbm, idx_v, d2_v, w_v, nch):
        def g_copy(k, b):
            return pltpu.make_async_copy(
                table_hbm.at[idx_v.at[pl.ds(k * CH, CH)]], bufs[b], sems[b])

        def s_copy(k, b):
            return pltpu.make_async_copy(bufs[b], acc.at[d2_v.at[k]], sems[b])

        for b in range(NBUF):
            g_copy(b, b).start()

        def quad(j, _):
            q0 = NBUF * j
            for b in range(NBUF):
                k = q0 + b
                g_copy(k, b).wait()
                scale(bufs[b], w_v, k * CH)
                s_copy(k, b).start(add=True)
                pb = (b - 1) % NBUF
                if b > 0:
                    s_copy(k - 1, pb).wait()

                    @pl.when(k + NBUF - 1 < nch)
                    def _():
                        g_copy(k + NBUF - 1, pb).start()
                else:
                    @pl.when(j > 0)
                    def _():
                        s_copy(k - 1, pb).wait()
                        g_copy(k + NBUF - 1, pb).start()
            return _

        lax.fori_loop(0, nch // NBUF, quad, None)
        s_copy(nch - 1, NBUF - 1).wait()

    def stage_copies(g, bset):
        sg, dg, ag = bset
        base = wid * ETT_W + g * GRP
        return (
            pltpu.make_async_copy(src_hbm.at[pl.ds(base, GRP)], sg, sem_st),
            pltpu.make_async_copy(alpha_hbm.at[pl.ds(base, GRP)], ag, sem_st),
            pltpu.make_async_copy(
                dst2_hbm.at[wid, pl.ds(g * GRP_CH, GRP_CH)], dg, sem_st),
        )

    def stage_start(g, bset):
        for cp in stage_copies(g, bset):
            cp.start()

    def stage_wait(g, bset):
        for cp in stage_copies(g, bset):
            cp.wait()

    stage_start(0, stage_bufs[0])
    stage_wait(0, stage_bufs[0])

    def gpair(j, _):
        g0 = 2 * j
        stage_start(g0 + 1, stage_bufs[1])
        sg, dg, ag = stage_bufs[0]
        run_chunks(h_hbm, sg, dg, ag, GRP_CH)
        stage_wait(g0 + 1, stage_bufs[1])

        @pl.when(g0 + 2 < N_GRP)
        def _():
            stage_start(g0 + 2, stage_bufs[0])

        sg, dg, ag = stage_bufs[1]
        run_chunks(h_hbm, sg, dg, ag, GRP_CH)

        @pl.when(g0 + 2 < N_GRP)
        def _():
            stage_wait(g0 + 2, stage_bufs[0])
        return _

    lax.fori_loop(0, N_GRP // 2, gpair, None)
    run_chunks(cu_hbm, usrc_v, udst2_v, beta_v, EU_CH)

    plsc.subcore_barrier()
    ocps = [pltpu.make_async_copy(
        acc.at[pl.ds(s * ROWS_W + j * CH, CH)],
        partial_hbm.at[c, pl.ds(s * ROWS_W + j * CH, CH)], sem_st)
        for j in range(ROWS_W // CH)]
    for cp in ocps:
        cp.start()
    for cp in ocps:
        cp.wait()


def _step(h, cu_t, alpha, beta, srcp, dst2, usrcp, udst2):
    mesh = plsc.VectorSubcoreMesh(core_axis_name="c", subcore_axis_name="s")
    f = functools.partial(
        pl.kernel,
        out_type=jax.ShapeDtypeStruct((NC, NP, DH), jnp.float32),
        mesh=mesh,
        compiler_params=pltpu.CompilerParams(needs_layout_passes=False),
        scratch_types=[
            pltpu.VMEM_SHARED((NP, DH), jnp.float32),
            pltpu.VMEM((GRP,), jnp.int32),
            pltpu.VMEM((GRP_CH, CH), jnp.int32),
            pltpu.VMEM((GRP,), jnp.float32),
            pltpu.VMEM((GRP,), jnp.int32),
            pltpu.VMEM((GRP_CH, CH), jnp.int32),
            pltpu.VMEM((GRP,), jnp.float32),
            pltpu.VMEM((EU_W,), jnp.int32),
            pltpu.VMEM((EU_CH, CH), jnp.int32),
            pltpu.VMEM((EU_W,), jnp.float32),
            pltpu.VMEM((CH, DH), jnp.float32),
            pltpu.VMEM((CH, DH), jnp.float32),
            pltpu.VMEM((CH, DH), jnp.float32),
            pltpu.VMEM((CH, DH), jnp.float32),
            pltpu.SemaphoreType.DMA,
            pltpu.SemaphoreType.DMA,
            pltpu.SemaphoreType.DMA,
            pltpu.SemaphoreType.DMA,
            pltpu.SemaphoreType.DMA,
        ],
    )(_step_body)
    return f(h, cu_t, alpha, beta, srcp, dst2, usrcp, udst2)


def _combine_body(p_ref, w_ref, b_ref, h_ref, xs_ref):
    h = p_ref[0] + p_ref[1]
    h_ref[...] = h
    xs_ref[...] = (jnp.sum(h * w_ref[0, :][None, :], axis=1, keepdims=True)
                   + b_ref[0, 0])


def _combine(partial, w_h2x_row, b_h2x):
    return pl.pallas_call(
        _combine_body,
        out_shape=(
            jax.ShapeDtypeStruct((NP, DH), jnp.float32),
            jax.ShapeDtypeStruct((NP, 1), jnp.float32),
        ),
    )(partial, w_h2x_row, b_h2x)


def kernel(pos, pos_u, edge_index_tt, u_src, u_dst, hist_x, history_u, us,
           W_alpha, b_alpha, W_beta, b_beta, W_hist, b_hist,
           W_c2h, b_c2h, W_h2x, b_h2x):
    f32 = jnp.float32
    pos_p = jnp.pad(pos, ((0, NP - N), (0, 0)))
    histx_p = jnp.pad(hist_x, ((0, NP - N), (0, 0)))
    u_full = jnp.concatenate([history_u, us], axis=1)

    src = edge_index_tt[0]
    dst = edge_index_tt[1]
    pad_tt = ETT_P - E_TT
    pad_ids = (jnp.arange(pad_tt, dtype=jnp.int32) % N)
    srcp = jnp.concatenate([src, pad_ids])
    dstp = jnp.concatenate([dst, pad_ids])
    pad_u = EU_P - E_U
    upad_s = (jnp.arange(pad_u, dtype=jnp.int32) % N_U)
    upad_d = (jnp.arange(pad_u, dtype=jnp.int32) % N)
    usrcp = jnp.concatenate([u_src, upad_s])
    udstp = jnp.concatenate([u_dst, upad_d])
    dst2 = dstp.reshape(NW, ETT_CH, CH)
    udst2 = udstp.reshape(NW, EU_CH, CH)

    w2 = jnp.repeat(jnp.concatenate(
        [W_alpha[:, 0], b_alpha, jnp.zeros((1,), f32),
         W_beta[:, 0], b_beta, jnp.zeros((1,), f32)]), 16)
    pcols = [jnp.asarray(pos_p[:, i]) for i in range(POS)]
    qcols = [jnp.asarray(pos_u[:, i]) for i in range(POS)]

    h0, cu = _prep(histx_p, W_hist, b_hist.reshape(1, DH), u_full, W_c2h,
                   b_c2h.reshape(1, DH))

    alpha, beta = _gates(pcols, qcols, w2, srcp, dstp, usrcp, udstp)

    w_row = W_h2x[:, 0].reshape(1, DH)
    b11 = b_h2x.reshape(1, 1)

    h = h0
    xs_cols = []
    for t in range(T):
        partial = _step(h, cu[t], alpha, beta, srcp, dst2, usrcp, udst2)
        h, xs_t = _combine(partial, w_row, b11)
        xs_cols.append(xs_t)

    xs = jnp.concatenate(xs_cols, axis=1)[:N]
    return xs


# speedup vs baseline: 11.0147x; 1.0682x over previous
import functools

import jax
import jax.numpy as jnp
from jax import lax
from jax.experimental import pallas as pl
from jax.experimental.pallas import tpu as pltpu
from jax.experimental.pallas import tpu_sc as plsc

N_G = 8000
N_C = 2000
N = N_G + N_C
NP = 10240
N_U = 2000
E_TT = 320000
E_U = 40000
POS = 3
HIST = 20
UH = 8
T = 4
DH = 128

NC = 2
NS = 16
NW = NC * NS
CH = 64
NBUF = 4

ETT_W = 10240
ETT_P = NW * ETT_W
ETT_CH = ETT_W // CH
EU_W = 1280
EU_P = NW * EU_W
EU_CH = EU_W // CH

ROWS_W = NP // NS
GRP = 1024
GRP_CH = GRP // CH
N_GRP = ETT_W // GRP


def _prep_body(histx_ref, whist_ref, bhist_ref, ufull_ref, wc2h_ref, bc2h_ref,
               h0_ref, cu_ref):
    h0_ref[...] = (jnp.dot(histx_ref[...], whist_ref[...],
                           preferred_element_type=jnp.float32)
                   + bhist_ref[0, :][None, :])
    for t in range(T):
        u_win = ufull_ref[:, t:t + UH]
        cu_ref[t] = (jnp.dot(u_win, wc2h_ref[...],
                             preferred_element_type=jnp.float32)
                     + bc2h_ref[0, :][None, :])


def _prep(histx_p, w_hist, b_hist, u_full, w_c2h, b_c2h):
    return pl.pallas_call(
        _prep_body,
        out_shape=(
            jax.ShapeDtypeStruct((NP, DH), jnp.float32),
            jax.ShapeDtypeStruct((T, N_U, DH), jnp.float32),
        ),
    )(histx_p, w_hist, b_hist, u_full, w_c2h, b_c2h)


def _tanh16(x):
    e = jnp.exp(-2.0 * jnp.abs(x))
    t = (1.0 - e) / (1.0 + e)
    return jnp.where(x < 0.0, -t, t)


def _gate_body(p0_hbm, p1_hbm, p2_hbm, q0_hbm, q1_hbm, q2_hbm, w_hbm,
               src_hbm, dst_hbm, usrc_hbm, udst_hbm,
               alpha_hbm, beta_hbm,
               gsa_v, gda_v, gsb_v, gdb_v,
               p0_v, p1_v, p2_v, q0_v, q1_v, q2_v, w_v,
               src_v, dst_v, usrc_v, udst_v, alpha_v, beta_v):
    c = lax.axis_index("c")
    s = lax.axis_index("s")
    wid = s * NC + c
    iota = lax.iota(jnp.int32, 16)

    pltpu.sync_copy(p0_hbm, p0_v)
    pltpu.sync_copy(p1_hbm, p1_v)
    pltpu.sync_copy(p2_hbm, p2_v)
    pltpu.sync_copy(q0_hbm, q0_v)
    pltpu.sync_copy(q1_hbm, q1_v)
    pltpu.sync_copy(q2_hbm, q2_v)
    pltpu.sync_copy(w_hbm, w_v)

    def wscal(r, i):
        return w_v[pl.ds((r * 8 + i) * 16, 16)]

    def node_tab(i, _):
        sl = pl.ds(i * 16, 16)
        p0 = p0_v[sl]
        p1 = p1_v[sl]
        p2 = p2_v[sl]
        gsa_v[sl] = (p0 * wscal(0, 0) + p1 * wscal(0, 1) + p2 * wscal(0, 2))
        gda_v[sl] = (p0 * wscal(0, 3) + p1 * wscal(0, 4) + p2 * wscal(0, 5)
                     + wscal(0, 6))
        gdb_v[sl] = (p0 * wscal(1, 3) + p1 * wscal(1, 4) + p2 * wscal(1, 5)
                     + wscal(1, 6))
        return _

    lax.fori_loop(0, NP // 16, node_tab, None)

    def unode_tab(i, _):
        sl = pl.ds(i * 16, 16)
        gsb_v[sl] = (q0_v[sl] * wscal(1, 0) + q1_v[sl] * wscal(1, 1)
                     + q2_v[sl] * wscal(1, 2))
        return _

    lax.fori_loop(0, N_U // 16, unode_tab, None)

    base = wid * ETT_W
    pltpu.sync_copy(src_hbm.at[pl.ds(base, ETT_W)], src_v)
    pltpu.sync_copy(dst_hbm.at[pl.ds(base, ETT_W)], dst_v)
    ubase = wid * EU_W
    pltpu.sync_copy(usrc_hbm.at[pl.ds(ubase, EU_W)], usrc_v)
    pltpu.sync_copy(udst_hbm.at[pl.ds(ubase, EU_W)], udst_v)

    def tt_step(i, _):
        si = src_v[pl.ds(i * 16, 16)]
        di = dst_v[pl.ds(i * 16, 16)]
        x = (plsc.load_gather(gsa_v, [si]) + plsc.load_gather(gda_v, [di]))
        a = 0.05 * _tanh16(x)
        gidx = base + i * 16 + iota
        alpha_v[pl.ds(i * 16, 16)] = jnp.where(gidx < E_TT, a, 0.0)
        return _

    lax.fori_loop(0, ETT_W // 16, tt_step, None)
    pltpu.sync_copy(alpha_v, alpha_hbm.at[pl.ds(base, ETT_W)])

    def u_step(i, _):
        si = usrc_v[pl.ds(i * 16, 16)]
        di = udst_v[pl.ds(i * 16, 16)]
        x = (plsc.load_gather(gsb_v, [si]) + plsc.load_gather(gdb_v, [di]))
        b = 0.05 * _tanh16(x)
        gidx = ubase + i * 16 + iota
        beta_v[pl.ds(i * 16, 16)] = jnp.where(gidx < E_U, b, 0.0)
        return _

    lax.fori_loop(0, EU_W // 16, u_step, None)
    pltpu.sync_copy(beta_v, beta_hbm.at[pl.ds(ubase, EU_W)])


def _gates(pcols, qcols, w2, srcp, dstp, usrcp, udstp):
    mesh = plsc.VectorSubcoreMesh(core_axis_name="c", subcore_axis_name="s")
    f = functools.partial(
        pl.kernel,
        out_type=(
            jax.ShapeDtypeStruct((ETT_P,), jnp.float32),
            jax.ShapeDtypeStruct((EU_P,), jnp.float32),
        ),
        mesh=mesh,
        compiler_params=pltpu.CompilerParams(needs_layout_passes=False),
        scratch_types=[
            pltpu.VMEM((NP,), jnp.float32),
            pltpu.VMEM((NP,), jnp.float32),
            pltpu.VMEM((N_U,), jnp.float32),
            pltpu.VMEM((NP,), jnp.float32),
            pltpu.VMEM((NP,), jnp.float32),
            pltpu.VMEM((NP,), jnp.float32),
            pltpu.VMEM((NP,), jnp.float32),
            pltpu.VMEM((N_U,), jnp.float32),
            pltpu.VMEM((N_U,), jnp.float32),
            pltpu.VMEM((N_U,), jnp.float32),
            pltpu.VMEM((256,), jnp.float32),
            pltpu.VMEM((ETT_W,), jnp.int32),
            pltpu.VMEM((ETT_W,), jnp.int32),
            pltpu.VMEM((EU_W,), jnp.int32),
            pltpu.VMEM((EU_W,), jnp.int32),
            pltpu.VMEM((ETT_W,), jnp.float32),
            pltpu.VMEM((EU_W,), jnp.float32),
        ],
    )(_gate_body)
    return f(*pcols, *qcols, w2, srcp, dstp, usrcp, udstp)


def _step_body(h_hbm, cu_hbm, alpha_hbm, beta_hbm,
               src_hbm, dst2_hbm, usrc_hbm, udst2_hbm,
               partial_hbm,
               acc, srcg_0, dstg_0, alphag_0, srcg_1, dstg_1, alphag_1,
               usrc_v, udst2_v, beta_v,
               rows_0, rows_1, rows_2, rows_3,
               sem_0, sem_1, sem_2, sem_3, sem_st):
    c = lax.axis_index("c")
    s = lax.axis_index("s")
    wid = s * NC + c
    bufs = (rows_0, rows_1, rows_2, rows_3)
    sems = (sem_0, sem_1, sem_2, sem_3)
    stage_bufs = ((srcg_0, dstg_0, alphag_0), (srcg_1, dstg_1, alphag_1))

    zero16 = jnp.zeros((16,), jnp.float32)

    def zrow(i, _):
        for g in range(DH // 16):
            rows_0[i, pl.ds(g * 16, 16)] = zero16
        return _

    lax.fori_loop(0, CH, zrow, None)
    zcps = [pltpu.make_async_copy(
        rows_0, acc.at[pl.ds(s * ROWS_W + j * CH, CH)], sem_st)
        for j in range(ROWS_W // CH)]
    for cp in zcps:
        cp.start()
    for cp in zcps:
        cp.wait()

    ubase = wid * EU_W
    pltpu.sync_copy(usrc_hbm.at[pl.ds(ubase, EU_W)], usrc_v)
    pltpu.sync_copy(beta_hbm.at[pl.ds(ubase, EU_W)], beta_v)
    pltpu.sync_copy(udst2_hbm.at[wid], udst2_v)

    plsc.subcore_barrier()

    def scale(buf, w_v, wbase):
        gdn = lax.GatherDimensionNumbers(
            offset_dims=(), collapsed_slice_dims=(0,), start_index_map=(0,))

        def edge16(m, _):
            av16 = w_v[pl.ds(wbase + m * 16, 16)]
            for q in range(16):
                a = lax.gather(av16, jnp.full((16, 1), q, jnp.int32), gdn,
                               (1,), mode=lax.GatherScatterMode.PROMISE_IN_BOUNDS)
                e = m * 16 + q
                for g in range(DH // 16):
                    buf[e, pl.ds(g * 16, 16)] = buf[e, pl.ds(g * 16, 16)] * a
            return _

        lax.fori_loop(0, CH // 16, edge16, None)

    def run_chunks(table_hbm, idx_v, d2_v, w_v, nch):
        def g_copy(k, b):
            return pltpu.make_async_copy(
                table_hbm.at[idx_v.at[pl.ds(k * CH, CH)]], bufs[b], sems[b])

        def s_copy(k, b):
            return pltpu.make_async_copy(bufs[b], acc.at[d2_v.at[k]], sems[b])

        for b in range(NBUF):
            g_copy(b, b).start()

        def quad(j, _):
            q0 = NBUF * j
            for b in range(NBUF):
                k = q0 + b
                g_copy(k, b).wait()
                scale(bufs[b], w_v, k * CH)
                s_copy(k, b).start(add=True)
                pb = (b - 1) % NBUF
                if b > 0:
                    s_copy(k - 1, pb).wait()

                    @pl.when(k + NBUF - 1 < nch)
                    def _():
                        g_copy(k + NBUF - 1, pb).start()
                else:
                    @pl.when(j > 0)
                    def _():
                        s_copy(k - 1, pb).wait()
                        g_copy(k + NBUF - 1, pb).start()
            return _

        lax.fori_loop(0, nch // NBUF, quad, None)
        s_copy(nch - 1, NBUF - 1).wait()

    def stage_copies(g, bset):
        sg, dg, ag = bset
        base = wid * ETT_W + g * GRP
        return (
            pltpu.make_async_copy(src_hbm.at[pl.ds(base, GRP)], sg, sem_st),
            pltpu.make_async_copy(alpha_hbm.at[pl.ds(base, GRP)], ag, sem_st),
            pltpu.make_async_copy(
                dst2_hbm.at[wid, pl.ds(g * GRP_CH, GRP_CH)], dg, sem_st),
        )

    def stage_start(g, bset):
        for cp in stage_copies(g, bset):
            cp.start()

    def stage_wait(g, bset):
        for cp in stage_copies(g, bset):
            cp.wait()

    stage_start(0, stage_bufs[0])
    stage_wait(0, stage_bufs[0])

    def gpair(j, _):
        g0 = 2 * j
        stage_start(g0 + 1, stage_bufs[1])
        sg, dg, ag = stage_bufs[0]
        run_chunks(h_hbm, sg, dg, ag, GRP_CH)
        stage_wait(g0 + 1, stage_bufs[1])

        @pl.when(g0 + 2 < N_GRP)
        def _():
            stage_start(g0 + 2, stage_bufs[0])

        sg, dg, ag = stage_bufs[1]
        run_chunks(h_hbm, sg, dg, ag, GRP_CH)

        @pl.when(g0 + 2 < N_GRP)
        def _():
            stage_wait(g0 + 2, stage_bufs[0])
        return _

    lax.fori_loop(0, N_GRP // 2, gpair, None)
    run_chunks(cu_hbm, usrc_v, udst2_v, beta_v, EU_CH)

    plsc.subcore_barrier()
    ocps = [pltpu.make_async_copy(
        acc.at[pl.ds(s * ROWS_W + j * CH, CH)],
        partial_hbm.at[c, pl.ds(s * ROWS_W + j * CH, CH)], sem_st)
        for j in range(ROWS_W // CH)]
    for cp in ocps:
        cp.start()
    for cp in ocps:
        cp.wait()


def _step(h, cu_t, alpha, beta, srcp, dst2, usrcp, udst2):
    mesh = plsc.VectorSubcoreMesh(core_axis_name="c", subcore_axis_name="s")
    f = functools.partial(
        pl.kernel,
        out_type=jax.ShapeDtypeStruct((NC, NP, DH), jnp.float32),
        mesh=mesh,
        compiler_params=pltpu.CompilerParams(needs_layout_passes=False),
        scratch_types=[
            pltpu.VMEM_SHARED((NP, DH), jnp.float32),
            pltpu.VMEM((GRP,), jnp.int32),
            pltpu.VMEM((GRP_CH, CH), jnp.int32),
            pltpu.VMEM((GRP,), jnp.float32),
            pltpu.VMEM((GRP,), jnp.int32),
            pltpu.VMEM((GRP_CH, CH), jnp.int32),
            pltpu.VMEM((GRP,), jnp.float32),
            pltpu.VMEM((EU_W,), jnp.int32),
            pltpu.VMEM((EU_CH, CH), jnp.int32),
            pltpu.VMEM((EU_W,), jnp.float32),
            pltpu.VMEM((CH, DH), jnp.float32),
            pltpu.VMEM((CH, DH), jnp.float32),
            pltpu.VMEM((CH, DH), jnp.float32),
            pltpu.VMEM((CH, DH), jnp.float32),
            pltpu.SemaphoreType.DMA,
            pltpu.SemaphoreType.DMA,
            pltpu.SemaphoreType.DMA,
            pltpu.SemaphoreType.DMA,
            pltpu.SemaphoreType.DMA,
        ],
    )(_step_body)
    return f(h, cu_t, alpha, beta, srcp, dst2, usrcp, udst2)


def _combine_body(p_ref, w_ref, b_ref, h_ref, xs_ref):
    h = p_ref[0] + p_ref[1]
    h_ref[...] = h
    xs_ref[...] = (jnp.sum(h * w_ref[0, :][None, :], axis=1, keepdims=True)
                   + b_ref[0, 0])


def _combine(partial, w_h2x_row, b_h2x):
    return pl.pallas_call(
        _combine_body,
        out_shape=(
            jax.ShapeDtypeStruct((NP, DH), jnp.float32),
            jax.ShapeDtypeStruct((NP, 1), jnp.float32),
        ),
    )(partial, w_h2x_row, b_h2x)


def kernel(pos, pos_u, edge_index_tt, u_src, u_dst, hist_x, history_u, us,
           W_alpha, b_alpha, W_beta, b_beta, W_hist, b_hist,
           W_c2h, b_c2h, W_h2x, b_h2x):
    f32 = jnp.float32
    pos_p = jnp.pad(pos, ((0, NP - N), (0, 0)))
    histx_p = jnp.pad(hist_x, ((0, NP - N), (0, 0)))
    u_full = jnp.concatenate([history_u, us], axis=1)

    src = edge_index_tt[0]
    dst = edge_index_tt[1]
    pad_tt = ETT_P - E_TT
    pad_ids = (jnp.arange(pad_tt, dtype=jnp.int32) % N)
    srcp = jnp.concatenate([src, pad_ids])
    dstp = jnp.concatenate([dst, pad_ids])
    pad_u = EU_P - E_U
    upad_s = (jnp.arange(pad_u, dtype=jnp.int32) % N_U)
    upad_d = (jnp.arange(pad_u, dtype=jnp.int32) % N)
    usrcp = jnp.concatenate([u_src, upad_s])
    udstp = jnp.concatenate([u_dst, upad_d])
    dst2 = dstp.reshape(NW, ETT_CH, CH)
    udst2 = udstp.reshape(NW, EU_CH, CH)

    w2 = jnp.repeat(jnp.concatenate(
        [W_alpha[:, 0], b_alpha, jnp.zeros((1,), f32),
         W_beta[:, 0], b_beta, jnp.zeros((1,), f32)]), 16)
    pcols = [jnp.asarray(pos_p[:, i]) for i in range(POS)]
    qcols = [jnp.asarray(pos_u[:, i]) for i in range(POS)]

    h0, cu = _prep(histx_p, W_hist, b_hist.reshape(1, DH), u_full, W_c2h,
                   b_c2h.reshape(1, DH))

    alpha, beta = _gates(pcols, qcols, w2, srcp, dstp, usrcp, udstp)

    w_row = W_h2x[:, 0].reshape(1, DH)
    b11 = b_h2x.reshape(1, 1)

    h = h0
    xs_cols = []
    for t in range(T):
        partial = _step(h, cu[t], alpha, beta, srcp, dst2, usrcp, udst2)
        h, xs_t = _combine(partial, w_row, b11)
        xs_cols.append(xs_t)

    xs = jnp.concatenate(xs_cols, axis=1)[:N]
    return xs
